# Initial kernel scaffold; baseline (speedup 1.0000x reference)
import jax
import jax.numpy as jnp
from jax.experimental import pallas as pl


def kernel(x, gate_w, gate_b, ew1, ew2, ew3, sw1, sw2, sw3):
    raise NotImplementedError("write your pallas kernel here")


# speedup vs baseline: 1.1345x; 1.1345x over previous
import functools

import jax
import jax.numpy as jnp
from jax import lax
from jax.experimental import pallas as pl
from jax.experimental.pallas import tpu as pltpu
from jax.experimental.pallas import tpu_sc as plsc

T = 2048
D = 2048
E = 8
K = 2
INTER = 1024
SH_INTER = 1024
P = T * K

TT = 128
NT = P // TT + E
R = NT * TT
NTP = 48

NC = 2
NS = 16
NW = NC * NS
LANES = 16

@functools.lru_cache(maxsize=None)
def _sc_mesh():
    return plsc.VectorSubcoreMesh(
        core_axis_name="c", subcore_axis_name="s",
        num_cores=NC, num_subcores=NS)


def _gate_body(x_ref, gw_ref, gb_ref, idx_ref, wt_ref):
    x = x_ref[...]
    gw = gw_ref[...]
    logits = lax.dot_general(x, gw, (((1,), (1,)), ((), ())),
                             preferred_element_type=jnp.float32)
    m = jnp.max(logits, axis=1, keepdims=True)
    ex = jnp.exp(logits - m)
    s = ex / jnp.sum(ex, axis=1, keepdims=True)
    b = s + gb_ref[...]
    iota = lax.broadcasted_iota(jnp.int32, s.shape, 1)
    v1 = jnp.max(b, axis=1, keepdims=True)
    i1 = jnp.min(jnp.where(b >= v1, iota, E), axis=1, keepdims=True)
    w1 = jnp.sum(jnp.where(iota == i1, s, 0.0), axis=1, keepdims=True)
    b2 = jnp.where(iota == i1, -jnp.inf, b)
    v2 = jnp.max(b2, axis=1, keepdims=True)
    i2 = jnp.min(jnp.where(b2 >= v2, iota, E), axis=1, keepdims=True)
    w2 = jnp.sum(jnp.where(iota == i2, s, 0.0), axis=1, keepdims=True)
    idx_ref[...] = jnp.concatenate([i1, i2], axis=1)
    wt_ref[...] = jnp.concatenate([w1, w2], axis=1)


def _gate(x, gate_w, gate_b):
    bt = 256
    return pl.pallas_call(
        _gate_body,
        grid=(T // bt,),
        in_specs=[
            pl.BlockSpec((bt, D), lambda i: (i, 0)),
            pl.BlockSpec((E, D), lambda i: (0, 0)),
            pl.BlockSpec((1, E), lambda i: (0, 0)),
        ],
        out_specs=[
            pl.BlockSpec((bt, K), lambda i: (i, 0)),
            pl.BlockSpec((bt, K), lambda i: (i, 0)),
        ],
        out_shape=[
            jax.ShapeDtypeStruct((T, K), jnp.int32),
            jax.ShapeDtypeStruct((T, K), jnp.float32),
        ],
    )(x, gate_w, gate_b.reshape(1, E))


def _permute(v, idx):
    return lax.gather(
        v, idx[:, None],
        lax.GatherDimensionNumbers(
            offset_dims=(), collapsed_slice_dims=(0,), start_index_map=(0,)),
        (1,), mode=lax.GatherScatterMode.PROMISE_IN_BOUNDS)


def _bcast_lane(v, e):
    return _permute(v, jnp.full((LANES,), e, jnp.int32))


def _incl_scan(s, ii):
    for d in (1, 2, 4, 8):
        g = _permute(s, jnp.maximum(ii - d, 0))
        s = s + jnp.where(ii >= d, g, 0)
    return s


def _dispatch_body(eidx_hbm, gwt_hbm, st_hbm, ws_hbm, pos_hbm, te_hbm,
                   e_v, g_v, st_v, ws_v, pos_v, te_v, stv_v):
    wid = lax.axis_index("s") * NC + lax.axis_index("c")

    @pl.when(wid == 0)
    def _():
        pltpu.sync_copy(eidx_hbm, e_v)
        pltpu.sync_copy(gwt_hbm, g_v)

        zf = jnp.zeros((LANES,), jnp.float32)
        zi = jnp.zeros((LANES,), jnp.int32)
        ii = lax.iota(jnp.int32, LANES)

        def init_body(i, _):
            st_v[pl.ds(i * LANES, LANES)] = (ii + i * LANES) & (T - 1)
            ws_v[pl.ds(i * LANES, LANES)] = zf
            return 0
        lax.fori_loop(0, R // LANES, init_body, 0)

        def rank_body(i, cnt):
            v = e_v[pl.ds(i * LANES, LANES)]
            rank = zi
            for e in range(E):
                m = v == e
                sc = _incl_scan(jnp.where(m, 1, 0), ii)
                ce = _bcast_lane(cnt, e)
                rank = jnp.where(m, ce + sc - 1, rank)
                cnt = cnt + jnp.where(ii == e, _bcast_lane(sc, LANES - 1), 0)
            pos_v[pl.ds(i * LANES, LANES)] = rank
            return cnt
        cnt = lax.fori_loop(0, P // LANES, rank_body, zi)

        tt_log = TT.bit_length() - 1
        padded = ((cnt + (TT - 1)) >> tt_log) << tt_log
        starts = _incl_scan(padded, ii) - padded
        stv_v[...] = starts

        def scat_body(i, _):
            v = e_v[pl.ds(i * LANES, LANES)]
            r = pos_v[pl.ds(i * LANES, LANES)]
            pos = plsc.load_gather(stv_v, [v]) + r
            pos_v[pl.ds(i * LANES, LANES)] = pos
            tok = (ii + i * LANES) >> 1
            plsc.store_scatter(st_v, [pos], tok)
            plsc.store_scatter(ws_v, [pos], g_v[pl.ds(i * LANES, LANES)])
            return 0
        lax.fori_loop(0, P // LANES, scat_body, 0)

        for i in range(NTP // LANES):
            rows = (ii + i * LANES) * TT
            acc = zi
            for e in range(1, E):
                acc = acc + jnp.where(rows >= _bcast_lane(starts, e), 1, 0)
            te_v[pl.ds(i * LANES, LANES)] = acc

        pltpu.sync_copy(st_v, st_hbm)
        pltpu.sync_copy(ws_v, ws_hbm)
        pltpu.sync_copy(pos_v, pos_hbm)
        pltpu.sync_copy(te_v, te_hbm)


@functools.lru_cache(maxsize=None)
def _dispatch_kernel():
    return pl.kernel(
        _dispatch_body,
        out_type=(
            jax.ShapeDtypeStruct((R,), jnp.int32),
            jax.ShapeDtypeStruct((R,), jnp.float32),
            jax.ShapeDtypeStruct((P,), jnp.int32),
            jax.ShapeDtypeStruct((NTP,), jnp.int32),
        ),
        mesh=_sc_mesh(),
        scratch_types=[
            pltpu.VMEM((P,), jnp.int32),
            pltpu.VMEM((P,), jnp.float32),
            pltpu.VMEM((R,), jnp.int32),
            pltpu.VMEM((R,), jnp.float32),
            pltpu.VMEM((P,), jnp.int32),
            pltpu.VMEM((NTP,), jnp.int32),
            pltpu.VMEM((LANES,), jnp.int32),
        ],
        compiler_params=pltpu.CompilerParams(needs_layout_passes=False),
    )


@functools.lru_cache(maxsize=None)
def _make_gather(nrows):
    per_w = nrows // NW
    chunk = 32
    nch = per_w // chunk

    @functools.partial(
        pl.kernel,
        out_type=jax.ShapeDtypeStruct((nrows, D), jnp.float32),
        mesh=_sc_mesh(),
        scratch_types=[
            pltpu.VMEM((chunk,), jnp.int32),
            pltpu.VMEM((chunk, D), jnp.float32),
            pltpu.SemaphoreType.DMA,
        ],
        compiler_params=pltpu.CompilerParams(needs_layout_passes=False),
    )
    def gather(table_hbm, idx_hbm, out_hbm, idx_v, rows_v, sem):
        wid = lax.axis_index("s") * NC + lax.axis_index("c")
        base = wid * per_w
        for ch in range(nch):
            off = base + ch * chunk
            pltpu.sync_copy(idx_hbm.at[pl.ds(off, chunk)], idx_v)
            pltpu.async_copy(table_hbm.at[idx_v], rows_v, sem).wait()
            pltpu.sync_copy(rows_v, out_hbm.at[pl.ds(off, chunk)])

    return gather


def _ffn_body(te_ref, xs_ref, ws_ref, w1_ref, w3_ref, w2_ref, out_ref):
    xv = xs_ref[...]
    a = lax.dot_general(xv, w1_ref[0], (((1,), (1,)), ((), ())),
                        preferred_element_type=jnp.float32)
    b = lax.dot_general(xv, w3_ref[0], (((1,), (1,)), ((), ())),
                        preferred_element_type=jnp.float32)
    h = (a * jax.nn.sigmoid(a)) * b
    o = lax.dot_general(h, w2_ref[0], (((1,), (1,)), ((), ())),
                        preferred_element_type=jnp.float32)
    out_ref[...] = o * ws_ref[...]


def _ffn(te, xs, ws, ew1, ew3, ew2):
    grid_spec = pltpu.PrefetchScalarGridSpec(
        num_scalar_prefetch=1,
        grid=(NT,),
        in_specs=[
            pl.BlockSpec((TT, D), lambda i, te: (i, 0)),
            pl.BlockSpec((TT, 1), lambda i, te: (i, 0)),
            pl.BlockSpec((1, INTER, D), lambda i, te: (te[i], 0, 0)),
            pl.BlockSpec((1, INTER, D), lambda i, te: (te[i], 0, 0)),
            pl.BlockSpec((1, D, INTER), lambda i, te: (te[i], 0, 0)),
        ],
        out_specs=pl.BlockSpec((TT, D), lambda i, te: (i, 0)),
    )
    return pl.pallas_call(
        _ffn_body,
        grid_spec=grid_spec,
        out_shape=jax.ShapeDtypeStruct((R, D), jnp.float32),
    )(te, xs, ws, ew1, ew3, ew2)


def _shared_body(x_ref, sw1_ref, sw3_ref, sw2_ref, yg_ref, out_ref):
    xv = x_ref[...]
    a = lax.dot_general(xv, sw1_ref[...], (((1,), (1,)), ((), ())),
                        preferred_element_type=jnp.float32)
    b = lax.dot_general(xv, sw3_ref[...], (((1,), (1,)), ((), ())),
                        preferred_element_type=jnp.float32)
    h = (a * jax.nn.sigmoid(a)) * b
    z = lax.dot_general(h, sw2_ref[...], (((1,), (1,)), ((), ())),
                        preferred_element_type=jnp.float32)
    out_ref[...] = z + yg_ref[:, 0, :] + yg_ref[:, 1, :]


def _shared_combine(x, sw1, sw3, sw2, yg):
    bt = 128
    return pl.pallas_call(
        _shared_body,
        grid=(T // bt,),
        in_specs=[
            pl.BlockSpec((bt, D), lambda i: (i, 0)),
            pl.BlockSpec((SH_INTER, D), lambda i: (0, 0)),
            pl.BlockSpec((SH_INTER, D), lambda i: (0, 0)),
            pl.BlockSpec((D, SH_INTER), lambda i: (0, 0)),
            pl.BlockSpec((bt, K, D), lambda i: (i, 0, 0)),
        ],
        out_specs=pl.BlockSpec((bt, D), lambda i: (i, 0)),
        out_shape=jax.ShapeDtypeStruct((T, D), jnp.float32),
    )(x, sw1, sw3, sw2, yg)


def kernel(x, gate_w, gate_b, ew1, ew2, ew3, sw1, sw2, sw3):
    eidx, gwt = _gate(x, gate_w, gate_b)
    st, ws, pos, te = _dispatch_kernel()(eidx.reshape(P), gwt.reshape(P))
    xs = _make_gather(R)(x, st)
    ys = _ffn(te, xs, ws.reshape(R, 1), ew1, ew3, ew2)
    yg = _make_gather(P)(ys, pos)
    out = _shared_combine(x, sw1, sw3, sw2, yg.reshape(T, K, D))
    return out.reshape(x.shape)


# speedup vs baseline: 1.1352x; 1.0006x over previous
import functools

import jax
import jax.numpy as jnp
from jax import lax
from jax.experimental import pallas as pl
from jax.experimental.pallas import tpu as pltpu
from jax.experimental.pallas import tpu_sc as plsc

T = 2048
D = 2048
E = 8
K = 2
INTER = 1024
SH_INTER = 1024
P = T * K

TT = 128
NT = P // TT + E
R = NT * TT
NTP = 48

NC = 2
NS = 16
NW = NC * NS
LANES = 16

@functools.lru_cache(maxsize=None)
def _sc_mesh():
    return plsc.VectorSubcoreMesh(
        core_axis_name="c", subcore_axis_name="s",
        num_cores=NC, num_subcores=NS)


def _gate_body(x_ref, gw_ref, gb_ref, idx_ref, wt_ref):
    x = x_ref[...]
    gw = gw_ref[...]
    logits = lax.dot_general(x, gw, (((1,), (1,)), ((), ())),
                             precision=lax.Precision.HIGHEST,
                             preferred_element_type=jnp.float32)
    m = jnp.max(logits, axis=1, keepdims=True)
    ex = jnp.exp(logits - m)
    s = ex / jnp.sum(ex, axis=1, keepdims=True)
    b = s + gb_ref[...]
    iota = lax.broadcasted_iota(jnp.int32, s.shape, 1)
    v1 = jnp.max(b, axis=1, keepdims=True)
    i1 = jnp.min(jnp.where(b >= v1, iota, E), axis=1, keepdims=True)
    w1 = jnp.sum(jnp.where(iota == i1, s, 0.0), axis=1, keepdims=True)
    b2 = jnp.where(iota == i1, -jnp.inf, b)
    v2 = jnp.max(b2, axis=1, keepdims=True)
    i2 = jnp.min(jnp.where(b2 >= v2, iota, E), axis=1, keepdims=True)
    w2 = jnp.sum(jnp.where(iota == i2, s, 0.0), axis=1, keepdims=True)
    idx_ref[...] = jnp.concatenate([i1, i2], axis=1)
    wt_ref[...] = jnp.concatenate([w1, w2], axis=1)


def _gate(x, gate_w, gate_b):
    bt = 256
    return pl.pallas_call(
        _gate_body,
        grid=(T // bt,),
        in_specs=[
            pl.BlockSpec((bt, D), lambda i: (i, 0)),
            pl.BlockSpec((E, D), lambda i: (0, 0)),
            pl.BlockSpec((1, E), lambda i: (0, 0)),
        ],
        out_specs=[
            pl.BlockSpec((bt, K), lambda i: (i, 0)),
            pl.BlockSpec((bt, K), lambda i: (i, 0)),
        ],
        out_shape=[
            jax.ShapeDtypeStruct((T, K), jnp.int32),
            jax.ShapeDtypeStruct((T, K), jnp.float32),
        ],
    )(x, gate_w, gate_b.reshape(1, E))


def _permute(v, idx):
    return lax.gather(
        v, idx[:, None],
        lax.GatherDimensionNumbers(
            offset_dims=(), collapsed_slice_dims=(0,), start_index_map=(0,)),
        (1,), mode=lax.GatherScatterMode.PROMISE_IN_BOUNDS)


def _bcast_lane(v, e):
    return _permute(v, jnp.full((LANES,), e, jnp.int32))


def _incl_scan(s, ii):
    for d in (1, 2, 4, 8):
        g = _permute(s, jnp.maximum(ii - d, 0))
        s = s + jnp.where(ii >= d, g, 0)
    return s


def _dispatch_body(eidx_hbm, gwt_hbm, st_hbm, ws_hbm, pos_hbm, te_hbm,
                   e_v, g_v, st_v, ws_v, pos_v, te_v, stv_v):
    wid = lax.axis_index("s") * NC + lax.axis_index("c")

    @pl.when(wid == 0)
    def _():
        pltpu.sync_copy(eidx_hbm, e_v)
        pltpu.sync_copy(gwt_hbm, g_v)

        zf = jnp.zeros((LANES,), jnp.float32)
        zi = jnp.zeros((LANES,), jnp.int32)
        ii = lax.iota(jnp.int32, LANES)

        def init_body(i, _):
            st_v[pl.ds(i * LANES, LANES)] = (ii + i * LANES) & (T - 1)
            ws_v[pl.ds(i * LANES, LANES)] = zf
            return 0
        lax.fori_loop(0, R // LANES, init_body, 0)

        def rank_body(i, cnt):
            v = e_v[pl.ds(i * LANES, LANES)]
            rank = zi
            for e in range(E):
                m = v == e
                sc = _incl_scan(jnp.where(m, 1, 0), ii)
                ce = _bcast_lane(cnt, e)
                rank = jnp.where(m, ce + sc - 1, rank)
                cnt = cnt + jnp.where(ii == e, _bcast_lane(sc, LANES - 1), 0)
            pos_v[pl.ds(i * LANES, LANES)] = rank
            return cnt
        cnt = lax.fori_loop(0, P // LANES, rank_body, zi)

        tt_log = TT.bit_length() - 1
        padded = ((cnt + (TT - 1)) >> tt_log) << tt_log
        starts = _incl_scan(padded, ii) - padded
        stv_v[...] = starts

        def scat_body(i, _):
            v = e_v[pl.ds(i * LANES, LANES)]
            r = pos_v[pl.ds(i * LANES, LANES)]
            pos = plsc.load_gather(stv_v, [v]) + r
            pos_v[pl.ds(i * LANES, LANES)] = pos
            tok = (ii + i * LANES) >> 1
            plsc.store_scatter(st_v, [pos], tok)
            plsc.store_scatter(ws_v, [pos], g_v[pl.ds(i * LANES, LANES)])
            return 0
        lax.fori_loop(0, P // LANES, scat_body, 0)

        tot = _bcast_lane(starts, E - 1) + _bcast_lane(padded, E - 1)
        for i in range(NTP // LANES):
            rows = (ii + i * LANES) * TT
            acc = zi
            for e in range(1, E):
                acc = acc + jnp.where(rows >= _bcast_lane(starts, e), 1, 0)
            acc = acc + jnp.where(rows >= tot, E, 0)
            te_v[pl.ds(i * LANES, LANES)] = acc

        pltpu.sync_copy(st_v, st_hbm)
        pltpu.sync_copy(ws_v, ws_hbm)
        pltpu.sync_copy(pos_v, pos_hbm)
        pltpu.sync_copy(te_v, te_hbm)


@functools.lru_cache(maxsize=None)
def _dispatch_kernel():
    return pl.kernel(
        _dispatch_body,
        out_type=(
            jax.ShapeDtypeStruct((R,), jnp.int32),
            jax.ShapeDtypeStruct((R,), jnp.float32),
            jax.ShapeDtypeStruct((P,), jnp.int32),
            jax.ShapeDtypeStruct((NTP,), jnp.int32),
        ),
        mesh=_sc_mesh(),
        scratch_types=[
            pltpu.VMEM((P,), jnp.int32),
            pltpu.VMEM((P,), jnp.float32),
            pltpu.VMEM((R,), jnp.int32),
            pltpu.VMEM((R,), jnp.float32),
            pltpu.VMEM((P,), jnp.int32),
            pltpu.VMEM((NTP,), jnp.int32),
            pltpu.VMEM((LANES,), jnp.int32),
        ],
        compiler_params=pltpu.CompilerParams(needs_layout_passes=False),
    )


@functools.lru_cache(maxsize=None)
def _make_gather(nrows):
    per_w = nrows // NW
    chunk = 32
    nch = per_w // chunk

    @functools.partial(
        pl.kernel,
        out_type=jax.ShapeDtypeStruct((nrows, D), jnp.float32),
        mesh=_sc_mesh(),
        scratch_types=[
            pltpu.VMEM((chunk,), jnp.int32),
            pltpu.VMEM((chunk, D), jnp.float32),
            pltpu.SemaphoreType.DMA,
        ],
        compiler_params=pltpu.CompilerParams(needs_layout_passes=False),
    )
    def gather(table_hbm, idx_hbm, out_hbm, idx_v, rows_v, sem):
        wid = lax.axis_index("s") * NC + lax.axis_index("c")
        base = wid * per_w
        for ch in range(nch):
            off = base + ch * chunk
            pltpu.sync_copy(idx_hbm.at[pl.ds(off, chunk)], idx_v)
            pltpu.async_copy(table_hbm.at[idx_v], rows_v, sem).wait()
            pltpu.sync_copy(rows_v, out_hbm.at[pl.ds(off, chunk)])

    return gather


def _ffn_body(te_ref, xs_ref, ws_ref, w1_ref, w3_ref, w2_ref, out_ref):
    i = pl.program_id(0)

    @pl.when(te_ref[i] < E)
    def _compute():
        xv = xs_ref[...]
        a = lax.dot_general(xv, w1_ref[0], (((1,), (1,)), ((), ())),
                            preferred_element_type=jnp.float32)
        b = lax.dot_general(xv, w3_ref[0], (((1,), (1,)), ((), ())),
                            preferred_element_type=jnp.float32)
        h = (a * jax.nn.sigmoid(a)) * b
        o = lax.dot_general(h, w2_ref[0], (((1,), (1,)), ((), ())),
                            preferred_element_type=jnp.float32)
        out_ref[...] = o * ws_ref[...]

    @pl.when(te_ref[i] >= E)
    def _skip():
        out_ref[...] = jnp.zeros_like(out_ref)


def _ffn(te, xs, ws, ew1, ew3, ew2):
    grid_spec = pltpu.PrefetchScalarGridSpec(
        num_scalar_prefetch=1,
        grid=(NT,),
        in_specs=[
            pl.BlockSpec((TT, D), lambda i, te: (i, 0)),
            pl.BlockSpec((TT, 1), lambda i, te: (i, 0)),
            pl.BlockSpec((1, INTER, D), lambda i, te: (te[i] & (E - 1), 0, 0)),
            pl.BlockSpec((1, INTER, D), lambda i, te: (te[i] & (E - 1), 0, 0)),
            pl.BlockSpec((1, D, INTER), lambda i, te: (te[i] & (E - 1), 0, 0)),
        ],
        out_specs=pl.BlockSpec((TT, D), lambda i, te: (i, 0)),
    )
    return pl.pallas_call(
        _ffn_body,
        grid_spec=grid_spec,
        out_shape=jax.ShapeDtypeStruct((R, D), jnp.float32),
    )(te, xs, ws, ew1, ew3, ew2)


def _shared_body(x_ref, sw1_ref, sw3_ref, sw2_ref, yg_ref, out_ref):
    xv = x_ref[...]
    a = lax.dot_general(xv, sw1_ref[...], (((1,), (1,)), ((), ())),
                        preferred_element_type=jnp.float32)
    b = lax.dot_general(xv, sw3_ref[...], (((1,), (1,)), ((), ())),
                        preferred_element_type=jnp.float32)
    h = (a * jax.nn.sigmoid(a)) * b
    z = lax.dot_general(h, sw2_ref[...], (((1,), (1,)), ((), ())),
                        preferred_element_type=jnp.float32)
    out_ref[...] = z + yg_ref[:, 0, :] + yg_ref[:, 1, :]


def _shared_combine(x, sw1, sw3, sw2, yg):
    bt = 128
    return pl.pallas_call(
        _shared_body,
        grid=(T // bt,),
        in_specs=[
            pl.BlockSpec((bt, D), lambda i: (i, 0)),
            pl.BlockSpec((SH_INTER, D), lambda i: (0, 0)),
            pl.BlockSpec((SH_INTER, D), lambda i: (0, 0)),
            pl.BlockSpec((D, SH_INTER), lambda i: (0, 0)),
            pl.BlockSpec((bt, K, D), lambda i: (i, 0, 0)),
        ],
        out_specs=pl.BlockSpec((bt, D), lambda i: (i, 0)),
        out_shape=jax.ShapeDtypeStruct((T, D), jnp.float32),
    )(x, sw1, sw3, sw2, yg)


def kernel(x, gate_w, gate_b, ew1, ew2, ew3, sw1, sw2, sw3):
    eidx, gwt = _gate(x, gate_w, gate_b)
    st, ws, pos, te = _dispatch_kernel()(eidx.reshape(P), gwt.reshape(P))
    xs = _make_gather(R)(x, st)
    ys = _ffn(te, xs, ws.reshape(R, 1), ew1, ew3, ew2)
    yg = _make_gather(P)(ys, pos)
    out = _shared_combine(x, sw1, sw3, sw2, yg.reshape(T, K, D))
    return out.reshape(x.shape)


# speedup vs baseline: 1.1576x; 1.0198x over previous
import functools

import jax
import jax.numpy as jnp
from jax import lax
from jax.experimental import pallas as pl
from jax.experimental.pallas import tpu as pltpu
from jax.experimental.pallas import tpu_sc as plsc

T = 2048
D = 2048
E = 8
K = 2
INTER = 1024
SH_INTER = 1024
P = T * K

TT = 128
NT = P // TT + E
R = NT * TT
NTP = 48

NC = 2
NS = 16
NW = NC * NS
LANES = 16

@functools.lru_cache(maxsize=None)
def _sc_mesh():
    return plsc.VectorSubcoreMesh(
        core_axis_name="c", subcore_axis_name="s",
        num_cores=NC, num_subcores=NS)


def _gate_body(x_ref, gw_ref, gb_ref, idx_ref, wt_ref):
    x = x_ref[...]
    gw = gw_ref[...]
    logits = lax.dot_general(x, gw, (((1,), (1,)), ((), ())),
                             preferred_element_type=jnp.float32)
    m = jnp.max(logits, axis=1, keepdims=True)
    ex = jnp.exp(logits - m)
    s = ex / jnp.sum(ex, axis=1, keepdims=True)
    b = s + gb_ref[...]
    iota = lax.broadcasted_iota(jnp.int32, s.shape, 1)
    v1 = jnp.max(b, axis=1, keepdims=True)
    i1 = jnp.min(jnp.where(b >= v1, iota, E), axis=1, keepdims=True)
    w1 = jnp.sum(jnp.where(iota == i1, s, 0.0), axis=1, keepdims=True)
    b2 = jnp.where(iota == i1, -jnp.inf, b)
    v2 = jnp.max(b2, axis=1, keepdims=True)
    i2 = jnp.min(jnp.where(b2 >= v2, iota, E), axis=1, keepdims=True)
    w2 = jnp.sum(jnp.where(iota == i2, s, 0.0), axis=1, keepdims=True)
    idx_ref[...] = jnp.concatenate([i1, i2], axis=1)
    wt_ref[...] = jnp.concatenate([w1, w2], axis=1)


def _gate(x, gate_w, gate_b):
    bt = 256
    return pl.pallas_call(
        _gate_body,
        grid=(T // bt,),
        in_specs=[
            pl.BlockSpec((bt, D), lambda i: (i, 0)),
            pl.BlockSpec((E, D), lambda i: (0, 0)),
            pl.BlockSpec((1, E), lambda i: (0, 0)),
        ],
        out_specs=[
            pl.BlockSpec((bt, K), lambda i: (i, 0)),
            pl.BlockSpec((bt, K), lambda i: (i, 0)),
        ],
        out_shape=[
            jax.ShapeDtypeStruct((T, K), jnp.int32),
            jax.ShapeDtypeStruct((T, K), jnp.float32),
        ],
    )(x, gate_w, gate_b.reshape(1, E))


def _permute(v, idx):
    return lax.gather(
        v, idx[:, None],
        lax.GatherDimensionNumbers(
            offset_dims=(), collapsed_slice_dims=(0,), start_index_map=(0,)),
        (1,), mode=lax.GatherScatterMode.PROMISE_IN_BOUNDS)


def _bcast_lane(v, e):
    return _permute(v, jnp.full((LANES,), e, jnp.int32))


def _incl_scan(s, ii):
    for d in (1, 2, 4, 8):
        g = _permute(s, jnp.maximum(ii - d, 0))
        s = s + jnp.where(ii >= d, g, 0)
    return s


def _dispatch_body(eidx_hbm, gwt_hbm, st_hbm, ws_hbm, pos_hbm, te_hbm,
                   e_v, g_v, st_v, ws_v, pos_v, te_v, stv_v):
    wid = lax.axis_index("s") * NC + lax.axis_index("c")

    @pl.when(wid == 0)
    def _():
        pltpu.sync_copy(eidx_hbm, e_v)
        pltpu.sync_copy(gwt_hbm, g_v)

        zf = jnp.zeros((LANES,), jnp.float32)
        zi = jnp.zeros((LANES,), jnp.int32)
        ii = lax.iota(jnp.int32, LANES)

        def init_body(i, _):
            st_v[pl.ds(i * LANES, LANES)] = (ii + i * LANES) & (T - 1)
            ws_v[pl.ds(i * LANES, LANES)] = zf
            return 0
        lax.fori_loop(0, R // LANES, init_body, 0)

        def rank_body(i, cnt):
            v = e_v[pl.ds(i * LANES, LANES)]
            rank = zi
            for e in range(E):
                m = v == e
                sc = _incl_scan(jnp.where(m, 1, 0), ii)
                ce = _bcast_lane(cnt, e)
                rank = jnp.where(m, ce + sc - 1, rank)
                cnt = cnt + jnp.where(ii == e, _bcast_lane(sc, LANES - 1), 0)
            pos_v[pl.ds(i * LANES, LANES)] = rank
            return cnt
        cnt = lax.fori_loop(0, P // LANES, rank_body, zi)

        tt_log = TT.bit_length() - 1
        padded = ((cnt + (TT - 1)) >> tt_log) << tt_log
        starts = _incl_scan(padded, ii) - padded
        stv_v[...] = starts

        def scat_body(i, _):
            v = e_v[pl.ds(i * LANES, LANES)]
            r = pos_v[pl.ds(i * LANES, LANES)]
            pos = plsc.load_gather(stv_v, [v]) + r
            pos_v[pl.ds(i * LANES, LANES)] = pos
            tok = (ii + i * LANES) >> 1
            plsc.store_scatter(st_v, [pos], tok)
            plsc.store_scatter(ws_v, [pos], g_v[pl.ds(i * LANES, LANES)])
            return 0
        lax.fori_loop(0, P // LANES, scat_body, 0)

        tot = _bcast_lane(starts, E - 1) + _bcast_lane(padded, E - 1)
        for i in range(NTP // LANES):
            rows = (ii + i * LANES) * TT
            acc = zi
            for e in range(1, E):
                acc = acc + jnp.where(rows >= _bcast_lane(starts, e), 1, 0)
            acc = acc + jnp.where(rows >= tot, E, 0)
            te_v[pl.ds(i * LANES, LANES)] = acc

        pltpu.sync_copy(st_v, st_hbm)
        pltpu.sync_copy(ws_v, ws_hbm)
        pltpu.sync_copy(pos_v, pos_hbm)
        pltpu.sync_copy(te_v, te_hbm)


@functools.lru_cache(maxsize=None)
def _dispatch_kernel():
    return pl.kernel(
        _dispatch_body,
        out_type=(
            jax.ShapeDtypeStruct((R,), jnp.int32),
            jax.ShapeDtypeStruct((R,), jnp.float32),
            jax.ShapeDtypeStruct((P,), jnp.int32),
            jax.ShapeDtypeStruct((NTP,), jnp.int32),
        ),
        mesh=_sc_mesh(),
        scratch_types=[
            pltpu.VMEM((P,), jnp.int32),
            pltpu.VMEM((P,), jnp.float32),
            pltpu.VMEM((R,), jnp.int32),
            pltpu.VMEM((R,), jnp.float32),
            pltpu.VMEM((P,), jnp.int32),
            pltpu.VMEM((NTP,), jnp.int32),
            pltpu.VMEM((LANES,), jnp.int32),
        ],
        compiler_params=pltpu.CompilerParams(needs_layout_passes=False),
    )


@functools.lru_cache(maxsize=None)
def _make_gather(nrows):
    per_w = nrows // NW
    chunk = 32
    nch = per_w // chunk

    @functools.partial(
        pl.kernel,
        out_type=jax.ShapeDtypeStruct((nrows, D), jnp.float32),
        mesh=_sc_mesh(),
        scratch_types=[
            pltpu.VMEM((chunk,), jnp.int32),
            pltpu.VMEM((chunk, D), jnp.float32),
            pltpu.SemaphoreType.DMA,
        ],
        compiler_params=pltpu.CompilerParams(needs_layout_passes=False),
    )
    def gather(table_hbm, idx_hbm, out_hbm, idx_v, rows_v, sem):
        wid = lax.axis_index("s") * NC + lax.axis_index("c")
        base = wid * per_w
        for ch in range(nch):
            off = base + ch * chunk
            pltpu.sync_copy(idx_hbm.at[pl.ds(off, chunk)], idx_v)
            pltpu.async_copy(table_hbm.at[idx_v], rows_v, sem).wait()
            pltpu.sync_copy(rows_v, out_hbm.at[pl.ds(off, chunk)])

    return gather


def _ffn_body(te_ref, xs_ref, ws_ref, w1_ref, w3_ref, w2_ref, out_ref):
    i = pl.program_id(0)

    @pl.when(te_ref[i] < E)
    def _compute():
        xv = xs_ref[...]
        a = lax.dot_general(xv, w1_ref[0], (((1,), (1,)), ((), ())),
                            preferred_element_type=jnp.float32)
        b = lax.dot_general(xv, w3_ref[0], (((1,), (1,)), ((), ())),
                            preferred_element_type=jnp.float32)
        h = (a * jax.nn.sigmoid(a)) * b
        o = lax.dot_general(h, w2_ref[0], (((1,), (1,)), ((), ())),
                            preferred_element_type=jnp.float32)
        out_ref[...] = o * ws_ref[...]

    @pl.when(te_ref[i] >= E)
    def _skip():
        out_ref[...] = jnp.zeros_like(out_ref)


def _ffn(te, xs, ws, ew1, ew3, ew2):
    grid_spec = pltpu.PrefetchScalarGridSpec(
        num_scalar_prefetch=1,
        grid=(NT,),
        in_specs=[
            pl.BlockSpec((TT, D), lambda i, te: (i, 0)),
            pl.BlockSpec((TT, 1), lambda i, te: (i, 0)),
            pl.BlockSpec((1, INTER, D), lambda i, te: (te[i] & (E - 1), 0, 0)),
            pl.BlockSpec((1, INTER, D), lambda i, te: (te[i] & (E - 1), 0, 0)),
            pl.BlockSpec((1, D, INTER), lambda i, te: (te[i] & (E - 1), 0, 0)),
        ],
        out_specs=pl.BlockSpec((TT, D), lambda i, te: (i, 0)),
    )
    return pl.pallas_call(
        _ffn_body,
        grid_spec=grid_spec,
        out_shape=jax.ShapeDtypeStruct((R, D), jnp.float32),
    )(te, xs, ws, ew1, ew3, ew2)


def _shared_body(x_ref, sw1_ref, sw3_ref, sw2_ref, yg_ref, out_ref):
    xv = x_ref[...]
    a = lax.dot_general(xv, sw1_ref[...], (((1,), (1,)), ((), ())),
                        preferred_element_type=jnp.float32)
    b = lax.dot_general(xv, sw3_ref[...], (((1,), (1,)), ((), ())),
                        preferred_element_type=jnp.float32)
    h = (a * jax.nn.sigmoid(a)) * b
    z = lax.dot_general(h, sw2_ref[...], (((1,), (1,)), ((), ())),
                        preferred_element_type=jnp.float32)
    out_ref[...] = z + yg_ref[:, 0, :] + yg_ref[:, 1, :]


def _shared_combine(x, sw1, sw3, sw2, yg):
    bt = 128
    return pl.pallas_call(
        _shared_body,
        grid=(T // bt,),
        in_specs=[
            pl.BlockSpec((bt, D), lambda i: (i, 0)),
            pl.BlockSpec((SH_INTER, D), lambda i: (0, 0)),
            pl.BlockSpec((SH_INTER, D), lambda i: (0, 0)),
            pl.BlockSpec((D, SH_INTER), lambda i: (0, 0)),
            pl.BlockSpec((bt, K, D), lambda i: (i, 0, 0)),
        ],
        out_specs=pl.BlockSpec((bt, D), lambda i: (i, 0)),
        out_shape=jax.ShapeDtypeStruct((T, D), jnp.float32),
    )(x, sw1, sw3, sw2, yg)


def kernel(x, gate_w, gate_b, ew1, ew2, ew3, sw1, sw2, sw3):
    eidx, gwt = _gate(x, gate_w, gate_b)
    st, ws, pos, te = _dispatch_kernel()(eidx.reshape(P), gwt.reshape(P))
    xs = _make_gather(R)(x, st)
    ys = _ffn(te, xs, ws.reshape(R, 1), ew1, ew3, ew2)
    yg = _make_gather(P)(ys, pos)
    out = _shared_combine(x, sw1, sw3, sw2, yg.reshape(T, K, D))
    return out.reshape(x.shape)


# speedup vs baseline: 1.1762x; 1.0161x over previous
import functools

import jax
import jax.numpy as jnp
from jax import lax
from jax.experimental import pallas as pl
from jax.experimental.pallas import tpu as pltpu
from jax.experimental.pallas import tpu_sc as plsc

T = 2048
D = 2048
E = 8
K = 2
INTER = 1024
SH_INTER = 1024
P = T * K

TT = 128
NT = P // TT + E
R = NT * TT
NTP = 48

NC = 2
NS = 16
NW = NC * NS
LANES = 16

@functools.lru_cache(maxsize=None)
def _sc_mesh():
    return plsc.VectorSubcoreMesh(
        core_axis_name="c", subcore_axis_name="s",
        num_cores=NC, num_subcores=NS)


def _gate_body(x_ref, gw_ref, gb_ref, idx_ref, wt_ref):
    x = x_ref[...]
    gw = gw_ref[...]
    logits = lax.dot_general(x, gw, (((1,), (1,)), ((), ())),
                             preferred_element_type=jnp.float32)
    m = jnp.max(logits, axis=1, keepdims=True)
    ex = jnp.exp(logits - m)
    s = ex / jnp.sum(ex, axis=1, keepdims=True)
    b = s + gb_ref[...]
    iota = lax.broadcasted_iota(jnp.int32, s.shape, 1)
    v1 = jnp.max(b, axis=1, keepdims=True)
    i1 = jnp.min(jnp.where(b >= v1, iota, E), axis=1, keepdims=True)
    w1 = jnp.sum(jnp.where(iota == i1, s, 0.0), axis=1, keepdims=True)
    b2 = jnp.where(iota == i1, -jnp.inf, b)
    v2 = jnp.max(b2, axis=1, keepdims=True)
    i2 = jnp.min(jnp.where(b2 >= v2, iota, E), axis=1, keepdims=True)
    w2 = jnp.sum(jnp.where(iota == i2, s, 0.0), axis=1, keepdims=True)
    idx_ref[...] = jnp.concatenate([i1, i2], axis=1)
    wt_ref[...] = jnp.concatenate([w1, w2], axis=1)


def _gate(x, gate_w, gate_b):
    bt = 256
    return pl.pallas_call(
        _gate_body,
        grid=(T // bt,),
        in_specs=[
            pl.BlockSpec((bt, D), lambda i: (i, 0)),
            pl.BlockSpec((E, D), lambda i: (0, 0)),
            pl.BlockSpec((1, E), lambda i: (0, 0)),
        ],
        out_specs=[
            pl.BlockSpec((bt, K), lambda i: (i, 0)),
            pl.BlockSpec((bt, K), lambda i: (i, 0)),
        ],
        out_shape=[
            jax.ShapeDtypeStruct((T, K), jnp.int32),
            jax.ShapeDtypeStruct((T, K), jnp.float32),
        ],
    )(x, gate_w, gate_b.reshape(1, E))


def _permute(v, idx):
    return lax.gather(
        v, idx[:, None],
        lax.GatherDimensionNumbers(
            offset_dims=(), collapsed_slice_dims=(0,), start_index_map=(0,)),
        (1,), mode=lax.GatherScatterMode.PROMISE_IN_BOUNDS)


def _bcast_lane(v, e):
    return _permute(v, jnp.full((LANES,), e, jnp.int32))


def _incl_scan(s, ii):
    for d in (1, 2, 4, 8):
        g = _permute(s, jnp.maximum(ii - d, 0))
        s = s + jnp.where(ii >= d, g, 0)
    return s


def _dispatch_body(eidx_hbm, gwt_hbm, st_hbm, ws_hbm, pos_hbm, te_hbm,
                   e_v, g_v, st_v, ws_v, pos_v, te_v, stv_v):
    wid = lax.axis_index("s") * NC + lax.axis_index("c")

    @pl.when(wid == 0)
    def _():
        pltpu.sync_copy(eidx_hbm, e_v)
        pltpu.sync_copy(gwt_hbm, g_v)

        zf = jnp.zeros((LANES,), jnp.float32)
        zi = jnp.zeros((LANES,), jnp.int32)
        ii = lax.iota(jnp.int32, LANES)

        def init_body(i, _):
            st_v[pl.ds(i * LANES, LANES)] = (ii + i * LANES) & (T - 1)
            ws_v[pl.ds(i * LANES, LANES)] = zf
            return 0
        lax.fori_loop(0, R // LANES, init_body, 0)

        def rank_body(i, cnt):
            v = e_v[pl.ds(i * LANES, LANES)]
            rank = zi
            for e in range(E):
                m = v == e
                sc = _incl_scan(jnp.where(m, 1, 0), ii)
                ce = _bcast_lane(cnt, e)
                rank = jnp.where(m, ce + sc - 1, rank)
                cnt = cnt + jnp.where(ii == e, _bcast_lane(sc, LANES - 1), 0)
            pos_v[pl.ds(i * LANES, LANES)] = rank
            return cnt
        cnt = lax.fori_loop(0, P // LANES, rank_body, zi)

        tt_log = TT.bit_length() - 1
        padded = ((cnt + (TT - 1)) >> tt_log) << tt_log
        starts = _incl_scan(padded, ii) - padded
        stv_v[...] = starts

        def scat_body(i, _):
            v = e_v[pl.ds(i * LANES, LANES)]
            r = pos_v[pl.ds(i * LANES, LANES)]
            pos = plsc.load_gather(stv_v, [v]) + r
            pos_v[pl.ds(i * LANES, LANES)] = pos
            tok = (ii + i * LANES) >> 1
            plsc.store_scatter(st_v, [pos], tok)
            plsc.store_scatter(ws_v, [pos], g_v[pl.ds(i * LANES, LANES)])
            return 0
        lax.fori_loop(0, P // LANES, scat_body, 0)

        tot = _bcast_lane(starts, E - 1) + _bcast_lane(padded, E - 1)
        for i in range(NTP // LANES):
            rows = (ii + i * LANES) * TT
            acc = zi
            for e in range(1, E):
                acc = acc + jnp.where(rows >= _bcast_lane(starts, e), 1, 0)
            acc = acc + jnp.where(rows >= tot, E, 0)
            te_v[pl.ds(i * LANES, LANES)] = acc

        pltpu.sync_copy(st_v, st_hbm)
        pltpu.sync_copy(ws_v, ws_hbm)
        pltpu.sync_copy(pos_v, pos_hbm)
        pltpu.sync_copy(te_v, te_hbm)


@functools.lru_cache(maxsize=None)
def _dispatch_kernel():
    return pl.kernel(
        _dispatch_body,
        out_type=(
            jax.ShapeDtypeStruct((R,), jnp.int32),
            jax.ShapeDtypeStruct((R,), jnp.float32),
            jax.ShapeDtypeStruct((P,), jnp.int32),
            jax.ShapeDtypeStruct((NTP,), jnp.int32),
        ),
        mesh=_sc_mesh(),
        scratch_types=[
            pltpu.VMEM((P,), jnp.int32),
            pltpu.VMEM((P,), jnp.float32),
            pltpu.VMEM((R,), jnp.int32),
            pltpu.VMEM((R,), jnp.float32),
            pltpu.VMEM((P,), jnp.int32),
            pltpu.VMEM((NTP,), jnp.int32),
            pltpu.VMEM((LANES,), jnp.int32),
        ],
        compiler_params=pltpu.CompilerParams(needs_layout_passes=False),
    )


@functools.lru_cache(maxsize=None)
def _make_gather(nrows):
    per_w = nrows // NW
    chunk = 32
    nch = per_w // chunk

    @functools.partial(
        pl.kernel,
        out_type=jax.ShapeDtypeStruct((nrows, D), jnp.float32),
        mesh=_sc_mesh(),
        scratch_types=[
            pltpu.VMEM((chunk,), jnp.int32),
            pltpu.VMEM((chunk, D), jnp.float32),
            pltpu.SemaphoreType.DMA,
        ],
        compiler_params=pltpu.CompilerParams(needs_layout_passes=False),
    )
    def gather(table_hbm, idx_hbm, out_hbm, idx_v, rows_v, sem):
        wid = lax.axis_index("s") * NC + lax.axis_index("c")
        base = wid * per_w
        for ch in range(nch):
            off = base + ch * chunk
            pltpu.sync_copy(idx_hbm.at[pl.ds(off, chunk)], idx_v)
            pltpu.async_copy(table_hbm.at[idx_v], rows_v, sem).wait()
            pltpu.sync_copy(rows_v, out_hbm.at[pl.ds(off, chunk)])

    return gather


def _ffn_body(te_ref, xs_ref, ws_ref, w1_ref, w3_ref, w2_ref, out_ref):
    i = pl.program_id(0)

    @pl.when(te_ref[i] < E)
    def _compute():
        xv = xs_ref[...]
        a = lax.dot_general(xv, w1_ref[0], (((1,), (1,)), ((), ())),
                            preferred_element_type=jnp.float32)
        b = lax.dot_general(xv, w3_ref[0], (((1,), (1,)), ((), ())),
                            preferred_element_type=jnp.float32)
        h = (a * jax.nn.sigmoid(a)) * b
        o = lax.dot_general(h, w2_ref[0], (((1,), (1,)), ((), ())),
                            preferred_element_type=jnp.float32)
        out_ref[...] = o * ws_ref[...]

    @pl.when(te_ref[i] >= E)
    def _skip():
        out_ref[...] = jnp.zeros_like(out_ref)


def _ffn(te, xs, ws, ew1, ew3, ew2):
    grid_spec = pltpu.PrefetchScalarGridSpec(
        num_scalar_prefetch=1,
        grid=(NT,),
        in_specs=[
            pl.BlockSpec((TT, D), lambda i, te: (i, 0)),
            pl.BlockSpec((TT, 1), lambda i, te: (i, 0)),
            pl.BlockSpec((1, INTER, D), lambda i, te: (te[i] & (E - 1), 0, 0)),
            pl.BlockSpec((1, INTER, D), lambda i, te: (te[i] & (E - 1), 0, 0)),
            pl.BlockSpec((1, D, INTER), lambda i, te: (te[i] & (E - 1), 0, 0)),
        ],
        out_specs=pl.BlockSpec((TT, D), lambda i, te: (i, 0)),
    )
    return pl.pallas_call(
        _ffn_body,
        grid_spec=grid_spec,
        out_shape=jax.ShapeDtypeStruct((R, D), jnp.float32),
    )(te, xs, ws, ew1, ew3, ew2)


def _shared_body(x_ref, sw1_ref, sw3_ref, sw2_ref, out_ref):
    xv = x_ref[...]
    a = lax.dot_general(xv, sw1_ref[...], (((1,), (1,)), ((), ())),
                        preferred_element_type=jnp.float32)
    b = lax.dot_general(xv, sw3_ref[...], (((1,), (1,)), ((), ())),
                        preferred_element_type=jnp.float32)
    h = (a * jax.nn.sigmoid(a)) * b
    out_ref[...] = lax.dot_general(h, sw2_ref[...], (((1,), (1,)), ((), ())),
                                   preferred_element_type=jnp.float32)


def _shared(x, sw1, sw3, sw2):
    bt = 128
    return pl.pallas_call(
        _shared_body,
        grid=(T // bt,),
        in_specs=[
            pl.BlockSpec((bt, D), lambda i: (i, 0)),
            pl.BlockSpec((SH_INTER, D), lambda i: (0, 0)),
            pl.BlockSpec((SH_INTER, D), lambda i: (0, 0)),
            pl.BlockSpec((D, SH_INTER), lambda i: (0, 0)),
        ],
        out_specs=pl.BlockSpec((bt, D), lambda i: (i, 0)),
        out_shape=jax.ShapeDtypeStruct((T, D), jnp.float32),
    )(x, sw1, sw3, sw2)


def _combine_body(z_ref, yg_ref, out_ref):
    out_ref[...] = z_ref[...] + yg_ref[:, 0, :] + yg_ref[:, 1, :]


def _combine(z, yg):
    bt = 256
    return pl.pallas_call(
        _combine_body,
        grid=(T // bt,),
        in_specs=[
            pl.BlockSpec((bt, D), lambda i: (i, 0)),
            pl.BlockSpec((bt, K, D), lambda i: (i, 0, 0)),
        ],
        out_specs=pl.BlockSpec((bt, D), lambda i: (i, 0)),
        out_shape=jax.ShapeDtypeStruct((T, D), jnp.float32),
    )(z, yg)


def kernel(x, gate_w, gate_b, ew1, ew2, ew3, sw1, sw2, sw3):
    eidx, gwt = _gate(x, gate_w, gate_b)
    st, ws, pos, te = _dispatch_kernel()(eidx.reshape(P), gwt.reshape(P))
    z = _shared(x, sw1, sw3, sw2)
    xs = _make_gather(R)(x, st)
    ys = _ffn(te, xs, ws.reshape(R, 1), ew1, ew3, ew2)
    yg = _make_gather(P)(ys, pos)
    out = _combine(z, yg.reshape(T, K, D))
    return out.reshape(x.shape)


# speedup vs baseline: 1.3793x; 1.1727x over previous
import functools

import jax
import jax.numpy as jnp
from jax import lax
from jax.experimental import pallas as pl
from jax.experimental.pallas import tpu as pltpu
from jax.experimental.pallas import tpu_sc as plsc

T = 2048
D = 2048
E = 8
K = 2
INTER = 1024
SH_INTER = 1024
P = T * K

TT = 256
NT = P // TT + E
R = NT * TT
NTP = 48

NC = 2
NS = 16
NW = NC * NS
LANES = 16

@functools.lru_cache(maxsize=None)
def _sc_mesh():
    return plsc.VectorSubcoreMesh(
        core_axis_name="c", subcore_axis_name="s",
        num_cores=NC, num_subcores=NS)


def _gate_body(x_ref, gw_ref, gb_ref, idx_ref, wt_ref):
    x = x_ref[...]
    gw = gw_ref[...]
    logits = lax.dot_general(x, gw, (((1,), (1,)), ((), ())),
                             preferred_element_type=jnp.float32)
    m = jnp.max(logits, axis=1, keepdims=True)
    ex = jnp.exp(logits - m)
    s = ex / jnp.sum(ex, axis=1, keepdims=True)
    b = s + gb_ref[...]
    iota = lax.broadcasted_iota(jnp.int32, s.shape, 1)
    v1 = jnp.max(b, axis=1, keepdims=True)
    i1 = jnp.min(jnp.where(b >= v1, iota, E), axis=1, keepdims=True)
    w1 = jnp.sum(jnp.where(iota == i1, s, 0.0), axis=1, keepdims=True)
    b2 = jnp.where(iota == i1, -jnp.inf, b)
    v2 = jnp.max(b2, axis=1, keepdims=True)
    i2 = jnp.min(jnp.where(b2 >= v2, iota, E), axis=1, keepdims=True)
    w2 = jnp.sum(jnp.where(iota == i2, s, 0.0), axis=1, keepdims=True)
    idx_ref[...] = jnp.concatenate([i1, i2], axis=1)
    wt_ref[...] = jnp.concatenate([w1, w2], axis=1)


def _gate(x, gate_w, gate_b):
    bt = 256
    return pl.pallas_call(
        _gate_body,
        grid=(T // bt,),
        in_specs=[
            pl.BlockSpec((bt, D), lambda i: (i, 0)),
            pl.BlockSpec((E, D), lambda i: (0, 0)),
            pl.BlockSpec((1, E), lambda i: (0, 0)),
        ],
        out_specs=[
            pl.BlockSpec((bt, K), lambda i: (i, 0)),
            pl.BlockSpec((bt, K), lambda i: (i, 0)),
        ],
        out_shape=[
            jax.ShapeDtypeStruct((T, K), jnp.int32),
            jax.ShapeDtypeStruct((T, K), jnp.float32),
        ],
    )(x, gate_w, gate_b.reshape(1, E))


def _permute(v, idx):
    return lax.gather(
        v, idx[:, None],
        lax.GatherDimensionNumbers(
            offset_dims=(), collapsed_slice_dims=(0,), start_index_map=(0,)),
        (1,), mode=lax.GatherScatterMode.PROMISE_IN_BOUNDS)


def _bcast_lane(v, e):
    return _permute(v, jnp.full((LANES,), e, jnp.int32))


def _incl_scan(s, ii):
    for d in (1, 2, 4, 8):
        g = _permute(s, jnp.maximum(ii - d, 0))
        s = s + jnp.where(ii >= d, g, 0)
    return s


def _dispatch_body(eidx_hbm, gwt_hbm, st_hbm, ws_hbm, pos_hbm, te_hbm,
                   e_v, g_v, st_v, ws_v, pos_v, te_v, stv_v):
    wid = lax.axis_index("s") * NC + lax.axis_index("c")

    @pl.when(wid == 0)
    def _():
        pltpu.sync_copy(eidx_hbm, e_v)
        pltpu.sync_copy(gwt_hbm, g_v)

        zf = jnp.zeros((LANES,), jnp.float32)
        zi = jnp.zeros((LANES,), jnp.int32)
        ii = lax.iota(jnp.int32, LANES)

        def init_body(i, _):
            st_v[pl.ds(i * LANES, LANES)] = (ii + i * LANES) & (T - 1)
            ws_v[pl.ds(i * LANES, LANES)] = zf
            return 0
        lax.fori_loop(0, R // LANES, init_body, 0)

        def rank_body(i, cnt):
            v = e_v[pl.ds(i * LANES, LANES)]
            rank = zi
            for e in range(E):
                m = v == e
                sc = _incl_scan(jnp.where(m, 1, 0), ii)
                ce = _bcast_lane(cnt, e)
                rank = jnp.where(m, ce + sc - 1, rank)
                cnt = cnt + jnp.where(ii == e, _bcast_lane(sc, LANES - 1), 0)
            pos_v[pl.ds(i * LANES, LANES)] = rank
            return cnt
        cnt = lax.fori_loop(0, P // LANES, rank_body, zi)

        tt_log = TT.bit_length() - 1
        padded = ((cnt + (TT - 1)) >> tt_log) << tt_log
        starts = _incl_scan(padded, ii) - padded
        stv_v[...] = starts

        def scat_body(i, _):
            v = e_v[pl.ds(i * LANES, LANES)]
            r = pos_v[pl.ds(i * LANES, LANES)]
            pos = plsc.load_gather(stv_v, [v]) + r
            pos_v[pl.ds(i * LANES, LANES)] = pos
            tok = (ii + i * LANES) >> 1
            plsc.store_scatter(st_v, [pos], tok)
            plsc.store_scatter(ws_v, [pos], g_v[pl.ds(i * LANES, LANES)])
            return 0
        lax.fori_loop(0, P // LANES, scat_body, 0)

        tot = _bcast_lane(starts, E - 1) + _bcast_lane(padded, E - 1)
        for i in range(NTP // LANES):
            rows = (ii + i * LANES) * TT
            acc = zi
            for e in range(1, E):
                acc = acc + jnp.where(rows >= _bcast_lane(starts, e), 1, 0)
            acc = acc + jnp.where(rows >= tot, E, 0)
            te_v[pl.ds(i * LANES, LANES)] = acc

        pltpu.sync_copy(st_v, st_hbm)
        pltpu.sync_copy(ws_v, ws_hbm)
        pltpu.sync_copy(pos_v, pos_hbm)
        pltpu.sync_copy(te_v, te_hbm)


@functools.lru_cache(maxsize=None)
def _dispatch_kernel():
    return pl.kernel(
        _dispatch_body,
        out_type=(
            jax.ShapeDtypeStruct((R,), jnp.int32),
            jax.ShapeDtypeStruct((R,), jnp.float32),
            jax.ShapeDtypeStruct((P,), jnp.int32),
            jax.ShapeDtypeStruct((NTP,), jnp.int32),
        ),
        mesh=_sc_mesh(),
        scratch_types=[
            pltpu.VMEM((P,), jnp.int32),
            pltpu.VMEM((P,), jnp.float32),
            pltpu.VMEM((R,), jnp.int32),
            pltpu.VMEM((R,), jnp.float32),
            pltpu.VMEM((P,), jnp.int32),
            pltpu.VMEM((NTP,), jnp.int32),
            pltpu.VMEM((LANES,), jnp.int32),
        ],
        compiler_params=pltpu.CompilerParams(needs_layout_passes=False),
    )


@functools.lru_cache(maxsize=None)
def _make_gather(nrows):
    per_w = nrows // NW
    chunk = 32
    nch = per_w // chunk

    @functools.partial(
        pl.kernel,
        out_type=jax.ShapeDtypeStruct((nrows, D), jnp.float32),
        mesh=_sc_mesh(),
        scratch_types=[
            pltpu.VMEM((chunk,), jnp.int32),
            pltpu.VMEM((chunk, D), jnp.float32),
            pltpu.SemaphoreType.DMA,
        ],
        compiler_params=pltpu.CompilerParams(needs_layout_passes=False),
    )
    def gather(table_hbm, idx_hbm, out_hbm, idx_v, rows_v, sem):
        wid = lax.axis_index("s") * NC + lax.axis_index("c")
        base = wid * per_w
        for ch in range(nch):
            off = base + ch * chunk
            pltpu.sync_copy(idx_hbm.at[pl.ds(off, chunk)], idx_v)
            pltpu.async_copy(table_hbm.at[idx_v], rows_v, sem).wait()
            pltpu.sync_copy(rows_v, out_hbm.at[pl.ds(off, chunk)])

    return gather


def _ffn_body(te_ref, xs_ref, ws_ref, w1_ref, w3_ref, w2_ref, out_ref):
    i = pl.program_id(0)

    @pl.when(te_ref[i] < E)
    def _compute():
        xv = xs_ref[...]
        a = lax.dot_general(xv, w1_ref[0], (((1,), (1,)), ((), ())),
                            preferred_element_type=jnp.float32)
        b = lax.dot_general(xv, w3_ref[0], (((1,), (1,)), ((), ())),
                            preferred_element_type=jnp.float32)
        h = (a * jax.nn.sigmoid(a)) * b
        o = lax.dot_general(h, w2_ref[0], (((1,), (1,)), ((), ())),
                            preferred_element_type=jnp.float32)
        out_ref[...] = o * ws_ref[...]

    @pl.when(te_ref[i] >= E)
    def _skip():
        out_ref[...] = jnp.zeros_like(out_ref)


def _ffn(te, xs, ws, ew1, ew3, ew2):
    grid_spec = pltpu.PrefetchScalarGridSpec(
        num_scalar_prefetch=1,
        grid=(NT,),
        in_specs=[
            pl.BlockSpec((TT, D), lambda i, te: (i, 0)),
            pl.BlockSpec((TT, 1), lambda i, te: (i, 0)),
            pl.BlockSpec((1, INTER, D), lambda i, te: (te[i] & (E - 1), 0, 0)),
            pl.BlockSpec((1, INTER, D), lambda i, te: (te[i] & (E - 1), 0, 0)),
            pl.BlockSpec((1, D, INTER), lambda i, te: (te[i] & (E - 1), 0, 0)),
        ],
        out_specs=pl.BlockSpec((TT, D), lambda i, te: (i, 0)),
    )
    return pl.pallas_call(
        _ffn_body,
        grid_spec=grid_spec,
        out_shape=jax.ShapeDtypeStruct((R, D), jnp.float32),
    )(te, xs, ws, ew1, ew3, ew2)


def _shared_body(x_ref, sw1_ref, sw3_ref, sw2_ref, out_ref):
    xv = x_ref[...]
    a = lax.dot_general(xv, sw1_ref[...], (((1,), (1,)), ((), ())),
                        preferred_element_type=jnp.float32)
    b = lax.dot_general(xv, sw3_ref[...], (((1,), (1,)), ((), ())),
                        preferred_element_type=jnp.float32)
    h = (a * jax.nn.sigmoid(a)) * b
    out_ref[...] = lax.dot_general(h, sw2_ref[...], (((1,), (1,)), ((), ())),
                                   preferred_element_type=jnp.float32)


def _shared(x, sw1, sw3, sw2):
    bt = 128
    return pl.pallas_call(
        _shared_body,
        grid=(T // bt,),
        in_specs=[
            pl.BlockSpec((bt, D), lambda i: (i, 0)),
            pl.BlockSpec((SH_INTER, D), lambda i: (0, 0)),
            pl.BlockSpec((SH_INTER, D), lambda i: (0, 0)),
            pl.BlockSpec((D, SH_INTER), lambda i: (0, 0)),
        ],
        out_specs=pl.BlockSpec((bt, D), lambda i: (i, 0)),
        out_shape=jax.ShapeDtypeStruct((T, D), jnp.float32),
    )(x, sw1, sw3, sw2)


def _combine_body(z_ref, yg_ref, out_ref):
    out_ref[...] = z_ref[...] + yg_ref[:, 0, :] + yg_ref[:, 1, :]


def _combine(z, yg):
    bt = 256
    return pl.pallas_call(
        _combine_body,
        grid=(T // bt,),
        in_specs=[
            pl.BlockSpec((bt, D), lambda i: (i, 0)),
            pl.BlockSpec((bt, K, D), lambda i: (i, 0, 0)),
        ],
        out_specs=pl.BlockSpec((bt, D), lambda i: (i, 0)),
        out_shape=jax.ShapeDtypeStruct((T, D), jnp.float32),
    )(z, yg)


def kernel(x, gate_w, gate_b, ew1, ew2, ew3, sw1, sw2, sw3):
    eidx, gwt = _gate(x, gate_w, gate_b)
    st, ws, pos, te = _dispatch_kernel()(eidx.reshape(P), gwt.reshape(P))
    z = _shared(x, sw1, sw3, sw2)
    xs = _make_gather(R)(x, st)
    ys = _ffn(te, xs, ws.reshape(R, 1), ew1, ew3, ew2)
    yg = _make_gather(P)(ys, pos)
    out = _combine(z, yg.reshape(T, K, D))
    return out.reshape(x.shape)


# speedup vs baseline: 1.4064x; 1.0196x over previous
import functools

import jax
import jax.numpy as jnp
from jax import lax
from jax.experimental import pallas as pl
from jax.experimental.pallas import tpu as pltpu
from jax.experimental.pallas import tpu_sc as plsc

T = 2048
D = 2048
E = 8
K = 2
INTER = 1024
SH_INTER = 1024
P = T * K

TT = 256
NT = P // TT + E
R = NT * TT
NTP = 48

NC = 2
NS = 16
NW = NC * NS
LANES = 16

@functools.lru_cache(maxsize=None)
def _sc_mesh():
    return plsc.VectorSubcoreMesh(
        core_axis_name="c", subcore_axis_name="s",
        num_cores=NC, num_subcores=NS)


def _gate_body(x_ref, gw_ref, gb_ref, idx_ref, wt_ref):
    x = x_ref[...]
    gw = gw_ref[...]
    logits = lax.dot_general(x, gw, (((1,), (1,)), ((), ())),
                             preferred_element_type=jnp.float32)
    m = jnp.max(logits, axis=1, keepdims=True)
    ex = jnp.exp(logits - m)
    s = ex / jnp.sum(ex, axis=1, keepdims=True)
    b = s + gb_ref[...]
    iota = lax.broadcasted_iota(jnp.int32, s.shape, 1)
    v1 = jnp.max(b, axis=1, keepdims=True)
    i1 = jnp.min(jnp.where(b >= v1, iota, E), axis=1, keepdims=True)
    w1 = jnp.sum(jnp.where(iota == i1, s, 0.0), axis=1, keepdims=True)
    b2 = jnp.where(iota == i1, -jnp.inf, b)
    v2 = jnp.max(b2, axis=1, keepdims=True)
    i2 = jnp.min(jnp.where(b2 >= v2, iota, E), axis=1, keepdims=True)
    w2 = jnp.sum(jnp.where(iota == i2, s, 0.0), axis=1, keepdims=True)
    idx_ref[...] = jnp.concatenate([i1, i2], axis=1)
    wt_ref[...] = jnp.concatenate([w1, w2], axis=1)


def _gate(x, gate_w, gate_b):
    bt = 256
    return pl.pallas_call(
        _gate_body,
        grid=(T // bt,),
        in_specs=[
            pl.BlockSpec((bt, D), lambda i: (i, 0)),
            pl.BlockSpec((E, D), lambda i: (0, 0)),
            pl.BlockSpec((1, E), lambda i: (0, 0)),
        ],
        out_specs=[
            pl.BlockSpec((bt, K), lambda i: (i, 0)),
            pl.BlockSpec((bt, K), lambda i: (i, 0)),
        ],
        out_shape=[
            jax.ShapeDtypeStruct((T, K), jnp.int32),
            jax.ShapeDtypeStruct((T, K), jnp.float32),
        ],
    )(x, gate_w, gate_b.reshape(1, E))


def _permute(v, idx):
    return lax.gather(
        v, idx[:, None],
        lax.GatherDimensionNumbers(
            offset_dims=(), collapsed_slice_dims=(0,), start_index_map=(0,)),
        (1,), mode=lax.GatherScatterMode.PROMISE_IN_BOUNDS)


def _bcast_lane(v, e):
    return _permute(v, jnp.full((LANES,), e, jnp.int32))


def _incl_scan(s, ii):
    for d in (1, 2, 4, 8):
        g = _permute(s, jnp.maximum(ii - d, 0))
        s = s + jnp.where(ii >= d, g, 0)
    return s


def _dispatch_body(eidx_hbm, gwt_hbm, st_hbm, ws_hbm, pos_hbm, te_hbm,
                   e_v, g_v, st_v, ws_v, pos_v, te_v, stv_v):
    wid = lax.axis_index("s") * NC + lax.axis_index("c")

    @pl.when(wid == 0)
    def _():
        pltpu.sync_copy(eidx_hbm, e_v)
        pltpu.sync_copy(gwt_hbm, g_v)

        zf = jnp.zeros((LANES,), jnp.float32)
        zi = jnp.zeros((LANES,), jnp.int32)
        ii = lax.iota(jnp.int32, LANES)

        def init_body(i, _):
            st_v[pl.ds(i * LANES, LANES)] = (ii + i * LANES) & (T - 1)
            ws_v[pl.ds(i * LANES, LANES)] = zf
            return 0
        lax.fori_loop(0, R // LANES, init_body, 0)

        def rank_body(i, cnt):
            v = e_v[pl.ds(i * LANES, LANES)]
            rank = zi
            for e in range(E):
                m = v == e
                sc = _incl_scan(jnp.where(m, 1, 0), ii)
                ce = _bcast_lane(cnt, e)
                rank = jnp.where(m, ce + sc - 1, rank)
                cnt = cnt + jnp.where(ii == e, _bcast_lane(sc, LANES - 1), 0)
            pos_v[pl.ds(i * LANES, LANES)] = rank
            return cnt
        cnt = lax.fori_loop(0, P // LANES, rank_body, zi)

        tt_log = TT.bit_length() - 1
        padded = ((cnt + (TT - 1)) >> tt_log) << tt_log
        starts = _incl_scan(padded, ii) - padded
        stv_v[...] = starts

        def scat_body(i, _):
            v = e_v[pl.ds(i * LANES, LANES)]
            r = pos_v[pl.ds(i * LANES, LANES)]
            pos = plsc.load_gather(stv_v, [v]) + r
            pos_v[pl.ds(i * LANES, LANES)] = pos
            tok = (ii + i * LANES) >> 1
            plsc.store_scatter(st_v, [pos], tok)
            plsc.store_scatter(ws_v, [pos], g_v[pl.ds(i * LANES, LANES)])
            return 0
        lax.fori_loop(0, P // LANES, scat_body, 0)

        tot = _bcast_lane(starts, E - 1) + _bcast_lane(padded, E - 1)
        for i in range(NTP // LANES):
            rows = (ii + i * LANES) * TT
            acc = zi
            for e in range(1, E):
                acc = acc + jnp.where(rows >= _bcast_lane(starts, e), 1, 0)
            acc = acc + jnp.where(rows >= tot, E, 0)
            te_v[pl.ds(i * LANES, LANES)] = acc

        pltpu.sync_copy(st_v, st_hbm)
        pltpu.sync_copy(ws_v, ws_hbm)
        pltpu.sync_copy(pos_v, pos_hbm)
        pltpu.sync_copy(te_v, te_hbm)


@functools.lru_cache(maxsize=None)
def _dispatch_kernel():
    return pl.kernel(
        _dispatch_body,
        out_type=(
            jax.ShapeDtypeStruct((R,), jnp.int32),
            jax.ShapeDtypeStruct((R,), jnp.float32),
            jax.ShapeDtypeStruct((P,), jnp.int32),
            jax.ShapeDtypeStruct((NTP,), jnp.int32),
        ),
        mesh=_sc_mesh(),
        scratch_types=[
            pltpu.VMEM((P,), jnp.int32),
            pltpu.VMEM((P,), jnp.float32),
            pltpu.VMEM((R,), jnp.int32),
            pltpu.VMEM((R,), jnp.float32),
            pltpu.VMEM((P,), jnp.int32),
            pltpu.VMEM((NTP,), jnp.int32),
            pltpu.VMEM((LANES,), jnp.int32),
        ],
        compiler_params=pltpu.CompilerParams(needs_layout_passes=False),
    )


@functools.lru_cache(maxsize=None)
def _make_gather(nrows):
    per_w = nrows // NW
    chunk = 16
    nch = per_w // chunk

    @functools.partial(
        pl.kernel,
        out_type=jax.ShapeDtypeStruct((nrows, D), jnp.float32),
        mesh=_sc_mesh(),
        scratch_types=[
            pltpu.VMEM((per_w,), jnp.int32),
            pltpu.VMEM((chunk, D), jnp.float32),
            pltpu.VMEM((chunk, D), jnp.float32),
            pltpu.SemaphoreType.DMA,
            pltpu.SemaphoreType.DMA,
            pltpu.SemaphoreType.DMA,
            pltpu.SemaphoreType.DMA,
        ],
        compiler_params=pltpu.CompilerParams(needs_layout_passes=False),
    )
    def gather(table_hbm, idx_hbm, out_hbm, idx_v, rows_a, rows_b,
               gsem_a, gsem_b, osem_a, osem_b):
        wid = lax.axis_index("s") * NC + lax.axis_index("c")
        base = wid * per_w
        pltpu.sync_copy(idx_hbm.at[pl.ds(base, per_w)], idx_v)
        bufs = (rows_a, rows_b)
        gsems = (gsem_a, gsem_b)
        osems = (osem_a, osem_b)
        gets = [None] * nch
        puts = [None] * nch
        gets[0] = pltpu.async_copy(
            table_hbm.at[idx_v.at[pl.ds(0, chunk)]], bufs[0], gsems[0])
        for ch in range(nch):
            if ch + 1 < nch:
                if ch >= 1:
                    puts[ch - 1].wait()
                gets[ch + 1] = pltpu.async_copy(
                    table_hbm.at[idx_v.at[pl.ds((ch + 1) * chunk, chunk)]],
                    bufs[(ch + 1) % 2], gsems[(ch + 1) % 2])
            gets[ch].wait()
            puts[ch] = pltpu.async_copy(
                bufs[ch % 2], out_hbm.at[pl.ds(base + ch * chunk, chunk)],
                osems[ch % 2])
        puts[nch - 2].wait()
        puts[nch - 1].wait()

    return gather


def _ffn_body(te_ref, xs_ref, ws_ref, w1_ref, w3_ref, w2_ref, out_ref):
    i = pl.program_id(0)

    @pl.when(te_ref[i] < E)
    def _compute():
        xv = xs_ref[...]
        a = lax.dot_general(xv, w1_ref[0], (((1,), (1,)), ((), ())),
                            preferred_element_type=jnp.float32)
        b = lax.dot_general(xv, w3_ref[0], (((1,), (1,)), ((), ())),
                            preferred_element_type=jnp.float32)
        h = (a * jax.nn.sigmoid(a)) * b
        o = lax.dot_general(h, w2_ref[0], (((1,), (1,)), ((), ())),
                            preferred_element_type=jnp.float32)
        out_ref[...] = o * ws_ref[...]

    @pl.when(te_ref[i] >= E)
    def _skip():
        out_ref[...] = jnp.zeros_like(out_ref)


def _ffn(te, xs, ws, ew1, ew3, ew2):
    grid_spec = pltpu.PrefetchScalarGridSpec(
        num_scalar_prefetch=1,
        grid=(NT,),
        in_specs=[
            pl.BlockSpec((TT, D), lambda i, te: (i, 0)),
            pl.BlockSpec((TT, 1), lambda i, te: (i, 0)),
            pl.BlockSpec((1, INTER, D), lambda i, te: (te[i] & (E - 1), 0, 0)),
            pl.BlockSpec((1, INTER, D), lambda i, te: (te[i] & (E - 1), 0, 0)),
            pl.BlockSpec((1, D, INTER), lambda i, te: (te[i] & (E - 1), 0, 0)),
        ],
        out_specs=pl.BlockSpec((TT, D), lambda i, te: (i, 0)),
    )
    return pl.pallas_call(
        _ffn_body,
        grid_spec=grid_spec,
        out_shape=jax.ShapeDtypeStruct((R, D), jnp.float32),
    )(te, xs, ws, ew1, ew3, ew2)


def _shared_body(x_ref, sw1_ref, sw3_ref, sw2_ref, out_ref):
    xv = x_ref[...]
    a = lax.dot_general(xv, sw1_ref[...], (((1,), (1,)), ((), ())),
                        preferred_element_type=jnp.float32)
    b = lax.dot_general(xv, sw3_ref[...], (((1,), (1,)), ((), ())),
                        preferred_element_type=jnp.float32)
    h = (a * jax.nn.sigmoid(a)) * b
    out_ref[...] = lax.dot_general(h, sw2_ref[...], (((1,), (1,)), ((), ())),
                                   preferred_element_type=jnp.float32)


def _shared(x, sw1, sw3, sw2):
    bt = 128
    return pl.pallas_call(
        _shared_body,
        grid=(T // bt,),
        in_specs=[
            pl.BlockSpec((bt, D), lambda i: (i, 0)),
            pl.BlockSpec((SH_INTER, D), lambda i: (0, 0)),
            pl.BlockSpec((SH_INTER, D), lambda i: (0, 0)),
            pl.BlockSpec((D, SH_INTER), lambda i: (0, 0)),
        ],
        out_specs=pl.BlockSpec((bt, D), lambda i: (i, 0)),
        out_shape=jax.ShapeDtypeStruct((T, D), jnp.float32),
    )(x, sw1, sw3, sw2)


def _combine_body(z_ref, yg_ref, out_ref):
    out_ref[...] = z_ref[...] + yg_ref[:, 0, :] + yg_ref[:, 1, :]


def _combine(z, yg):
    bt = 256
    return pl.pallas_call(
        _combine_body,
        grid=(T // bt,),
        in_specs=[
            pl.BlockSpec((bt, D), lambda i: (i, 0)),
            pl.BlockSpec((bt, K, D), lambda i: (i, 0, 0)),
        ],
        out_specs=pl.BlockSpec((bt, D), lambda i: (i, 0)),
        out_shape=jax.ShapeDtypeStruct((T, D), jnp.float32),
    )(z, yg)


def kernel(x, gate_w, gate_b, ew1, ew2, ew3, sw1, sw2, sw3):
    eidx, gwt = _gate(x, gate_w, gate_b)
    st, ws, pos, te = _dispatch_kernel()(eidx.reshape(P), gwt.reshape(P))
    z = _shared(x, sw1, sw3, sw2)
    xs = _make_gather(R)(x, st)
    ys = _ffn(te, xs, ws.reshape(R, 1), ew1, ew3, ew2)
    yg = _make_gather(P)(ys, pos)
    out = _combine(z, yg.reshape(T, K, D))
    return out.reshape(x.shape)


# speedup vs baseline: 1.5725x; 1.1181x over previous
import functools

import jax
import jax.numpy as jnp
from jax import lax
from jax.experimental import pallas as pl
from jax.experimental.pallas import tpu as pltpu
from jax.experimental.pallas import tpu_sc as plsc

T = 2048
D = 2048
E = 8
K = 2
INTER = 1024
SH_INTER = 1024
P = T * K

TT = 256
NT = P // TT + E
R = NT * TT
NTP = 48

NC = 2
NS = 16
NW = NC * NS
LANES = 16

@functools.lru_cache(maxsize=None)
def _sc_mesh():
    return plsc.VectorSubcoreMesh(
        core_axis_name="c", subcore_axis_name="s",
        num_cores=NC, num_subcores=NS)


def _gate_body(x_ref, gw_ref, gb_ref, idx_ref, wt_ref):
    x = x_ref[...]
    gw = gw_ref[...]
    logits = lax.dot_general(x, gw, (((1,), (1,)), ((), ())),
                             preferred_element_type=jnp.float32)
    m = jnp.max(logits, axis=1, keepdims=True)
    ex = jnp.exp(logits - m)
    s = ex / jnp.sum(ex, axis=1, keepdims=True)
    b = s + gb_ref[...]
    iota = lax.broadcasted_iota(jnp.int32, s.shape, 1)
    v1 = jnp.max(b, axis=1, keepdims=True)
    i1 = jnp.min(jnp.where(b >= v1, iota, E), axis=1, keepdims=True)
    w1 = jnp.sum(jnp.where(iota == i1, s, 0.0), axis=1, keepdims=True)
    b2 = jnp.where(iota == i1, -jnp.inf, b)
    v2 = jnp.max(b2, axis=1, keepdims=True)
    i2 = jnp.min(jnp.where(b2 >= v2, iota, E), axis=1, keepdims=True)
    w2 = jnp.sum(jnp.where(iota == i2, s, 0.0), axis=1, keepdims=True)
    idx_ref[...] = jnp.concatenate([i1, i2], axis=1)
    wt_ref[...] = jnp.concatenate([w1, w2], axis=1)


def _gate(x, gate_w, gate_b):
    bt = 512
    return pl.pallas_call(
        _gate_body,
        grid=(T // bt,),
        in_specs=[
            pl.BlockSpec((bt, D), lambda i: (i, 0)),
            pl.BlockSpec((E, D), lambda i: (0, 0)),
            pl.BlockSpec((1, E), lambda i: (0, 0)),
        ],
        out_specs=[
            pl.BlockSpec((bt, K), lambda i: (i, 0)),
            pl.BlockSpec((bt, K), lambda i: (i, 0)),
        ],
        out_shape=[
            jax.ShapeDtypeStruct((T, K), jnp.int32),
            jax.ShapeDtypeStruct((T, K), jnp.float32),
        ],
    )(x, gate_w, gate_b.reshape(1, E))


def _permute(v, idx):
    return lax.gather(
        v, idx[:, None],
        lax.GatherDimensionNumbers(
            offset_dims=(), collapsed_slice_dims=(0,), start_index_map=(0,)),
        (1,), mode=lax.GatherScatterMode.PROMISE_IN_BOUNDS)


def _bcast_lane(v, e):
    return _permute(v, jnp.full((LANES,), e, jnp.int32))


def _incl_scan(s, ii):
    for d in (1, 2, 4, 8):
        g = _permute(s, jnp.maximum(ii - d, 0))
        s = s + jnp.where(ii >= d, g, 0)
    return s


def _dispatch_body(eidx_hbm, gwt_hbm, st_hbm, ws_hbm, pos_hbm, te_hbm,
                   e_v, g_v, st_v, ws_v, pos_v, te_v, stv_v):
    wid = lax.axis_index("s") * NC + lax.axis_index("c")

    @pl.when(wid == 0)
    def _():
        pltpu.sync_copy(eidx_hbm, e_v)
        pltpu.sync_copy(gwt_hbm, g_v)

        zf = jnp.zeros((LANES,), jnp.float32)
        zi = jnp.zeros((LANES,), jnp.int32)
        ii = lax.iota(jnp.int32, LANES)

        def init_body(i, _):
            st_v[pl.ds(i * LANES, LANES)] = (ii + i * LANES) & (T - 1)
            ws_v[pl.ds(i * LANES, LANES)] = zf
            return 0
        lax.fori_loop(0, R // LANES, init_body, 0)

        def rank_body(i, cnt):
            v = e_v[pl.ds(i * LANES, LANES)]
            rank = zi
            for e in range(E):
                m = v == e
                sc = _incl_scan(jnp.where(m, 1, 0), ii)
                ce = _bcast_lane(cnt, e)
                rank = jnp.where(m, ce + sc - 1, rank)
                cnt = cnt + jnp.where(ii == e, _bcast_lane(sc, LANES - 1), 0)
            pos_v[pl.ds(i * LANES, LANES)] = rank
            return cnt
        cnt = lax.fori_loop(0, P // LANES, rank_body, zi)

        tt_log = TT.bit_length() - 1
        padded = ((cnt + (TT - 1)) >> tt_log) << tt_log
        starts = _incl_scan(padded, ii) - padded
        stv_v[...] = starts

        def scat_body(i, _):
            v = e_v[pl.ds(i * LANES, LANES)]
            r = pos_v[pl.ds(i * LANES, LANES)]
            pos = plsc.load_gather(stv_v, [v]) + r
            pos_v[pl.ds(i * LANES, LANES)] = pos
            tok = (ii + i * LANES) >> 1
            plsc.store_scatter(st_v, [pos], tok)
            plsc.store_scatter(ws_v, [pos], g_v[pl.ds(i * LANES, LANES)])
            return 0
        lax.fori_loop(0, P // LANES, scat_body, 0)

        tot = _bcast_lane(starts, E - 1) + _bcast_lane(padded, E - 1)
        for i in range(NTP // LANES):
            rows = (ii + i * LANES) * TT
            acc = zi
            for e in range(1, E):
                acc = acc + jnp.where(rows >= _bcast_lane(starts, e), 1, 0)
            acc = acc + jnp.where(rows >= tot, E, 0)
            te_v[pl.ds(i * LANES, LANES)] = acc

        pltpu.sync_copy(st_v, st_hbm)
        pltpu.sync_copy(ws_v, ws_hbm)
        pltpu.sync_copy(pos_v, pos_hbm)
        pltpu.sync_copy(te_v, te_hbm)


@functools.lru_cache(maxsize=None)
def _dispatch_kernel():
    return pl.kernel(
        _dispatch_body,
        out_type=(
            jax.ShapeDtypeStruct((R,), jnp.int32),
            jax.ShapeDtypeStruct((R,), jnp.float32),
            jax.ShapeDtypeStruct((P,), jnp.int32),
            jax.ShapeDtypeStruct((NTP,), jnp.int32),
        ),
        mesh=_sc_mesh(),
        scratch_types=[
            pltpu.VMEM((P,), jnp.int32),
            pltpu.VMEM((P,), jnp.float32),
            pltpu.VMEM((R,), jnp.int32),
            pltpu.VMEM((R,), jnp.float32),
            pltpu.VMEM((P,), jnp.int32),
            pltpu.VMEM((NTP,), jnp.int32),
            pltpu.VMEM((LANES,), jnp.int32),
        ],
        compiler_params=pltpu.CompilerParams(needs_layout_passes=False),
    )


@functools.lru_cache(maxsize=None)
def _make_gather(nrows):
    per_w = nrows // NW
    chunk = 16
    nch = per_w // chunk

    @functools.partial(
        pl.kernel,
        out_type=jax.ShapeDtypeStruct((nrows, D), jnp.float32),
        mesh=_sc_mesh(),
        scratch_types=[
            pltpu.VMEM((per_w,), jnp.int32),
            pltpu.VMEM((chunk, D), jnp.float32),
            pltpu.VMEM((chunk, D), jnp.float32),
            pltpu.SemaphoreType.DMA,
            pltpu.SemaphoreType.DMA,
            pltpu.SemaphoreType.DMA,
            pltpu.SemaphoreType.DMA,
        ],
        compiler_params=pltpu.CompilerParams(needs_layout_passes=False),
    )
    def gather(table_hbm, idx_hbm, out_hbm, idx_v, rows_a, rows_b,
               gsem_a, gsem_b, osem_a, osem_b):
        wid = lax.axis_index("s") * NC + lax.axis_index("c")
        base = wid * per_w
        pltpu.sync_copy(idx_hbm.at[pl.ds(base, per_w)], idx_v)
        bufs = (rows_a, rows_b)
        gsems = (gsem_a, gsem_b)
        osems = (osem_a, osem_b)
        gets = [None] * nch
        puts = [None] * nch
        gets[0] = pltpu.async_copy(
            table_hbm.at[idx_v.at[pl.ds(0, chunk)]], bufs[0], gsems[0])
        for ch in range(nch):
            if ch + 1 < nch:
                if ch >= 1:
                    puts[ch - 1].wait()
                gets[ch + 1] = pltpu.async_copy(
                    table_hbm.at[idx_v.at[pl.ds((ch + 1) * chunk, chunk)]],
                    bufs[(ch + 1) % 2], gsems[(ch + 1) % 2])
            gets[ch].wait()
            puts[ch] = pltpu.async_copy(
                bufs[ch % 2], out_hbm.at[pl.ds(base + ch * chunk, chunk)],
                osems[ch % 2])
        puts[nch - 2].wait()
        puts[nch - 1].wait()

    return gather


def _ffn_body(te_ref, xs_ref, ws_ref, w1_ref, w3_ref, w2_ref, out_ref):
    i = pl.program_id(0)

    @pl.when(te_ref[i] < E)
    def _compute():
        xv = xs_ref[...]
        a = lax.dot_general(xv, w1_ref[0], (((1,), (1,)), ((), ())),
                            preferred_element_type=jnp.float32)
        b = lax.dot_general(xv, w3_ref[0], (((1,), (1,)), ((), ())),
                            preferred_element_type=jnp.float32)
        h = (a * jax.nn.sigmoid(a)) * b
        o = lax.dot_general(h, w2_ref[0], (((1,), (1,)), ((), ())),
                            preferred_element_type=jnp.float32)
        out_ref[...] = o * ws_ref[...]

    @pl.when(te_ref[i] >= E)
    def _skip():
        out_ref[...] = jnp.zeros_like(out_ref)


def _ffn(te, xs, ws, ew1, ew3, ew2):
    grid_spec = pltpu.PrefetchScalarGridSpec(
        num_scalar_prefetch=1,
        grid=(NT,),
        in_specs=[
            pl.BlockSpec((TT, D), lambda i, te: (i, 0)),
            pl.BlockSpec((TT, 1), lambda i, te: (i, 0)),
            pl.BlockSpec((1, INTER, D), lambda i, te: (te[i] & (E - 1), 0, 0)),
            pl.BlockSpec((1, INTER, D), lambda i, te: (te[i] & (E - 1), 0, 0)),
            pl.BlockSpec((1, D, INTER), lambda i, te: (te[i] & (E - 1), 0, 0)),
        ],
        out_specs=pl.BlockSpec((TT, D), lambda i, te: (i, 0)),
    )
    return pl.pallas_call(
        _ffn_body,
        grid_spec=grid_spec,
        out_shape=jax.ShapeDtypeStruct((R, D), jnp.float32),
    )(te, xs, ws, ew1, ew3, ew2)


def _shared_body(x_ref, sw1_ref, sw3_ref, sw2_ref, out_ref):
    xv = x_ref[...]
    a = lax.dot_general(xv, sw1_ref[...], (((1,), (1,)), ((), ())),
                        preferred_element_type=jnp.float32)
    b = lax.dot_general(xv, sw3_ref[...], (((1,), (1,)), ((), ())),
                        preferred_element_type=jnp.float32)
    h = (a * jax.nn.sigmoid(a)) * b
    out_ref[...] = lax.dot_general(h, sw2_ref[...], (((1,), (1,)), ((), ())),
                                   preferred_element_type=jnp.float32)


def _shared(x, sw1, sw3, sw2):
    bt = 256
    return pl.pallas_call(
        _shared_body,
        grid=(T // bt,),
        in_specs=[
            pl.BlockSpec((bt, D), lambda i: (i, 0)),
            pl.BlockSpec((SH_INTER, D), lambda i: (0, 0)),
            pl.BlockSpec((SH_INTER, D), lambda i: (0, 0)),
            pl.BlockSpec((D, SH_INTER), lambda i: (0, 0)),
        ],
        out_specs=pl.BlockSpec((bt, D), lambda i: (i, 0)),
        out_shape=jax.ShapeDtypeStruct((T, D), jnp.float32),
    )(x, sw1, sw3, sw2)


def _combine_body(z_ref, yg_ref, out_ref):
    out_ref[...] = z_ref[...] + yg_ref[:, 0, :] + yg_ref[:, 1, :]


def _combine(z, yg):
    bt = 512
    return pl.pallas_call(
        _combine_body,
        grid=(T // bt,),
        in_specs=[
            pl.BlockSpec((bt, D), lambda i: (i, 0)),
            pl.BlockSpec((bt, K, D), lambda i: (i, 0, 0)),
        ],
        out_specs=pl.BlockSpec((bt, D), lambda i: (i, 0)),
        out_shape=jax.ShapeDtypeStruct((T, D), jnp.float32),
    )(z, yg)


def kernel(x, gate_w, gate_b, ew1, ew2, ew3, sw1, sw2, sw3):
    eidx, gwt = _gate(x, gate_w, gate_b)
    st, ws, pos, te = _dispatch_kernel()(eidx.reshape(P), gwt.reshape(P))
    z = _shared(x, sw1, sw3, sw2)
    xs = _make_gather(R)(x, st)
    ys = _ffn(te, xs, ws.reshape(R, 1), ew1, ew3, ew2)
    yg = _make_gather(P)(ys, pos)
    out = _combine(z, yg.reshape(T, K, D))
    return out.reshape(x.shape)


# speedup vs baseline: 1.5844x; 1.0076x over previous
import functools

import jax
import jax.numpy as jnp
from jax import lax
from jax.experimental import pallas as pl
from jax.experimental.pallas import tpu as pltpu
from jax.experimental.pallas import tpu_sc as plsc

T = 2048
D = 2048
E = 8
K = 2
INTER = 1024
SH_INTER = 1024
P = T * K

TT = 256
NT = P // TT + E
R = NT * TT
NTP = 48

NC = 2
NS = 16
NW = NC * NS
LANES = 16

@functools.lru_cache(maxsize=None)
def _sc_mesh():
    return plsc.VectorSubcoreMesh(
        core_axis_name="c", subcore_axis_name="s",
        num_cores=NC, num_subcores=NS)


def _gate_body(x_ref, gw_ref, gb_ref, idx_ref, wt_ref):
    x = x_ref[...]
    gw = gw_ref[...]
    logits = lax.dot_general(x, gw, (((1,), (1,)), ((), ())),
                             preferred_element_type=jnp.float32)
    m = jnp.max(logits, axis=1, keepdims=True)
    ex = jnp.exp(logits - m)
    s = ex / jnp.sum(ex, axis=1, keepdims=True)
    b = s + gb_ref[...]
    iota = lax.broadcasted_iota(jnp.int32, s.shape, 1)
    v1 = jnp.max(b, axis=1, keepdims=True)
    i1 = jnp.min(jnp.where(b >= v1, iota, E), axis=1, keepdims=True)
    w1 = jnp.sum(jnp.where(iota == i1, s, 0.0), axis=1, keepdims=True)
    b2 = jnp.where(iota == i1, -jnp.inf, b)
    v2 = jnp.max(b2, axis=1, keepdims=True)
    i2 = jnp.min(jnp.where(b2 >= v2, iota, E), axis=1, keepdims=True)
    w2 = jnp.sum(jnp.where(iota == i2, s, 0.0), axis=1, keepdims=True)
    idx_ref[...] = jnp.concatenate([i1, i2], axis=1)
    wt_ref[...] = jnp.concatenate([w1, w2], axis=1)


def _gate(x, gate_w, gate_b):
    bt = 1024
    return pl.pallas_call(
        _gate_body,
        grid=(T // bt,),
        in_specs=[
            pl.BlockSpec((bt, D), lambda i: (i, 0)),
            pl.BlockSpec((E, D), lambda i: (0, 0)),
            pl.BlockSpec((1, E), lambda i: (0, 0)),
        ],
        out_specs=[
            pl.BlockSpec((bt, K), lambda i: (i, 0)),
            pl.BlockSpec((bt, K), lambda i: (i, 0)),
        ],
        out_shape=[
            jax.ShapeDtypeStruct((T, K), jnp.int32),
            jax.ShapeDtypeStruct((T, K), jnp.float32),
        ],
    )(x, gate_w, gate_b.reshape(1, E))


def _permute(v, idx):
    return lax.gather(
        v, idx[:, None],
        lax.GatherDimensionNumbers(
            offset_dims=(), collapsed_slice_dims=(0,), start_index_map=(0,)),
        (1,), mode=lax.GatherScatterMode.PROMISE_IN_BOUNDS)


def _bcast_lane(v, e):
    return _permute(v, jnp.full((LANES,), e, jnp.int32))


def _incl_scan(s, ii):
    for d in (1, 2, 4, 8):
        g = _permute(s, jnp.maximum(ii - d, 0))
        s = s + jnp.where(ii >= d, g, 0)
    return s


def _dispatch_body(eidx_hbm, gwt_hbm, st_hbm, ws_hbm, pos_hbm, te_hbm,
                   e_v, g_v, st_v, ws_v, pos_v, te_v, stv_v):
    wid = lax.axis_index("s") * NC + lax.axis_index("c")

    @pl.when(wid == 0)
    def _():
        pltpu.sync_copy(eidx_hbm, e_v)
        pltpu.sync_copy(gwt_hbm, g_v)

        zf = jnp.zeros((LANES,), jnp.float32)
        zi = jnp.zeros((LANES,), jnp.int32)
        ii = lax.iota(jnp.int32, LANES)

        def init_body(i, _):
            st_v[pl.ds(i * LANES, LANES)] = (ii + i * LANES) & (T - 1)
            ws_v[pl.ds(i * LANES, LANES)] = zf
            return 0
        lax.fori_loop(0, R // LANES, init_body, 0)

        def rank_body(i, cnt):
            v = e_v[pl.ds(i * LANES, LANES)]
            rank = zi
            for e in range(E):
                m = v == e
                sc = _incl_scan(jnp.where(m, 1, 0), ii)
                ce = _bcast_lane(cnt, e)
                rank = jnp.where(m, ce + sc - 1, rank)
                cnt = cnt + jnp.where(ii == e, _bcast_lane(sc, LANES - 1), 0)
            pos_v[pl.ds(i * LANES, LANES)] = rank
            return cnt
        cnt = lax.fori_loop(0, P // LANES, rank_body, zi)

        tt_log = TT.bit_length() - 1
        padded = ((cnt + (TT - 1)) >> tt_log) << tt_log
        starts = _incl_scan(padded, ii) - padded
        stv_v[...] = starts

        def scat_body(i, _):
            v = e_v[pl.ds(i * LANES, LANES)]
            r = pos_v[pl.ds(i * LANES, LANES)]
            pos = plsc.load_gather(stv_v, [v]) + r
            pos_v[pl.ds(i * LANES, LANES)] = pos
            tok = (ii + i * LANES) >> 1
            plsc.store_scatter(st_v, [pos], tok)
            plsc.store_scatter(ws_v, [pos], g_v[pl.ds(i * LANES, LANES)])
            return 0
        lax.fori_loop(0, P // LANES, scat_body, 0)

        tot = _bcast_lane(starts, E - 1) + _bcast_lane(padded, E - 1)
        for i in range(NTP // LANES):
            rows = (ii + i * LANES) * TT
            acc = zi
            for e in range(1, E):
                acc = acc + jnp.where(rows >= _bcast_lane(starts, e), 1, 0)
            acc = acc + jnp.where(rows >= tot, E, 0)
            te_v[pl.ds(i * LANES, LANES)] = acc

        pltpu.sync_copy(st_v, st_hbm)
        pltpu.sync_copy(ws_v, ws_hbm)
        pltpu.sync_copy(pos_v, pos_hbm)
        pltpu.sync_copy(te_v, te_hbm)


@functools.lru_cache(maxsize=None)
def _dispatch_kernel():
    return pl.kernel(
        _dispatch_body,
        out_type=(
            jax.ShapeDtypeStruct((R,), jnp.int32),
            jax.ShapeDtypeStruct((R,), jnp.float32),
            jax.ShapeDtypeStruct((P,), jnp.int32),
            jax.ShapeDtypeStruct((NTP,), jnp.int32),
        ),
        mesh=_sc_mesh(),
        scratch_types=[
            pltpu.VMEM((P,), jnp.int32),
            pltpu.VMEM((P,), jnp.float32),
            pltpu.VMEM((R,), jnp.int32),
            pltpu.VMEM((R,), jnp.float32),
            pltpu.VMEM((P,), jnp.int32),
            pltpu.VMEM((NTP,), jnp.int32),
            pltpu.VMEM((LANES,), jnp.int32),
        ],
        compiler_params=pltpu.CompilerParams(needs_layout_passes=False),
    )


@functools.lru_cache(maxsize=None)
def _make_gather(nrows):
    per_w = nrows // NW
    chunk = 16
    nch = per_w // chunk

    @functools.partial(
        pl.kernel,
        out_type=jax.ShapeDtypeStruct((nrows, D), jnp.float32),
        mesh=_sc_mesh(),
        scratch_types=[
            pltpu.VMEM((per_w,), jnp.int32),
            pltpu.VMEM((chunk, D), jnp.float32),
            pltpu.VMEM((chunk, D), jnp.float32),
            pltpu.SemaphoreType.DMA,
            pltpu.SemaphoreType.DMA,
            pltpu.SemaphoreType.DMA,
            pltpu.SemaphoreType.DMA,
        ],
        compiler_params=pltpu.CompilerParams(needs_layout_passes=False),
    )
    def gather(table_hbm, idx_hbm, out_hbm, idx_v, rows_a, rows_b,
               gsem_a, gsem_b, osem_a, osem_b):
        wid = lax.axis_index("s") * NC + lax.axis_index("c")
        base = wid * per_w
        pltpu.sync_copy(idx_hbm.at[pl.ds(base, per_w)], idx_v)
        bufs = (rows_a, rows_b)
        gsems = (gsem_a, gsem_b)
        osems = (osem_a, osem_b)
        gets = [None] * nch
        puts = [None] * nch
        gets[0] = pltpu.async_copy(
            table_hbm.at[idx_v.at[pl.ds(0, chunk)]], bufs[0], gsems[0])
        for ch in range(nch):
            if ch + 1 < nch:
                if ch >= 1:
                    puts[ch - 1].wait()
                gets[ch + 1] = pltpu.async_copy(
                    table_hbm.at[idx_v.at[pl.ds((ch + 1) * chunk, chunk)]],
                    bufs[(ch + 1) % 2], gsems[(ch + 1) % 2])
            gets[ch].wait()
            puts[ch] = pltpu.async_copy(
                bufs[ch % 2], out_hbm.at[pl.ds(base + ch * chunk, chunk)],
                osems[ch % 2])
        puts[nch - 2].wait()
        puts[nch - 1].wait()

    return gather


def _ffn_body(te_ref, xs_ref, ws_ref, w1_ref, w3_ref, w2_ref, out_ref):
    i = pl.program_id(0)

    @pl.when(te_ref[i] < E)
    def _compute():
        xv = xs_ref[...]
        a = lax.dot_general(xv, w1_ref[0], (((1,), (1,)), ((), ())),
                            preferred_element_type=jnp.float32)
        b = lax.dot_general(xv, w3_ref[0], (((1,), (1,)), ((), ())),
                            preferred_element_type=jnp.float32)
        h = (a * jax.nn.sigmoid(a)) * b
        o = lax.dot_general(h, w2_ref[0], (((1,), (1,)), ((), ())),
                            preferred_element_type=jnp.float32)
        out_ref[...] = o * ws_ref[...]

    @pl.when(te_ref[i] >= E)
    def _skip():
        out_ref[...] = jnp.zeros_like(out_ref)


def _ffn(te, xs, ws, ew1, ew3, ew2):
    grid_spec = pltpu.PrefetchScalarGridSpec(
        num_scalar_prefetch=1,
        grid=(NT,),
        in_specs=[
            pl.BlockSpec((TT, D), lambda i, te: (i, 0)),
            pl.BlockSpec((TT, 1), lambda i, te: (i, 0)),
            pl.BlockSpec((1, INTER, D), lambda i, te: (te[i] & (E - 1), 0, 0)),
            pl.BlockSpec((1, INTER, D), lambda i, te: (te[i] & (E - 1), 0, 0)),
            pl.BlockSpec((1, D, INTER), lambda i, te: (te[i] & (E - 1), 0, 0)),
        ],
        out_specs=pl.BlockSpec((TT, D), lambda i, te: (i, 0)),
    )
    return pl.pallas_call(
        _ffn_body,
        grid_spec=grid_spec,
        out_shape=jax.ShapeDtypeStruct((R, D), jnp.float32),
    )(te, xs, ws, ew1, ew3, ew2)


def _shared_body(x_ref, sw1_ref, sw3_ref, sw2_ref, out_ref):
    xv = x_ref[...]
    a = lax.dot_general(xv, sw1_ref[...], (((1,), (1,)), ((), ())),
                        preferred_element_type=jnp.float32)
    b = lax.dot_general(xv, sw3_ref[...], (((1,), (1,)), ((), ())),
                        preferred_element_type=jnp.float32)
    h = (a * jax.nn.sigmoid(a)) * b
    out_ref[...] = lax.dot_general(h, sw2_ref[...], (((1,), (1,)), ((), ())),
                                   preferred_element_type=jnp.float32)


def _shared(x, sw1, sw3, sw2):
    bt = 512
    return pl.pallas_call(
        _shared_body,
        grid=(T // bt,),
        in_specs=[
            pl.BlockSpec((bt, D), lambda i: (i, 0)),
            pl.BlockSpec((SH_INTER, D), lambda i: (0, 0)),
            pl.BlockSpec((SH_INTER, D), lambda i: (0, 0)),
            pl.BlockSpec((D, SH_INTER), lambda i: (0, 0)),
        ],
        out_specs=pl.BlockSpec((bt, D), lambda i: (i, 0)),
        out_shape=jax.ShapeDtypeStruct((T, D), jnp.float32),
    )(x, sw1, sw3, sw2)


def _combine_body(z_ref, yg_ref, out_ref):
    out_ref[...] = z_ref[...] + yg_ref[:, 0, :] + yg_ref[:, 1, :]


def _combine(z, yg):
    bt = 512
    return pl.pallas_call(
        _combine_body,
        grid=(T // bt,),
        in_specs=[
            pl.BlockSpec((bt, D), lambda i: (i, 0)),
            pl.BlockSpec((bt, K, D), lambda i: (i, 0, 0)),
        ],
        out_specs=pl.BlockSpec((bt, D), lambda i: (i, 0)),
        out_shape=jax.ShapeDtypeStruct((T, D), jnp.float32),
    )(z, yg)


def kernel(x, gate_w, gate_b, ew1, ew2, ew3, sw1, sw2, sw3):
    eidx, gwt = _gate(x, gate_w, gate_b)
    st, ws, pos, te = _dispatch_kernel()(eidx.reshape(P), gwt.reshape(P))
    z = _shared(x, sw1, sw3, sw2)
    xs = _make_gather(R)(x, st)
    ys = _ffn(te, xs, ws.reshape(R, 1), ew1, ew3, ew2)
    yg = _make_gather(P)(ys, pos)
    out = _combine(z, yg.reshape(T, K, D))
    return out.reshape(x.shape)


# speedup vs baseline: 1.7523x; 1.1059x over previous
import functools

import jax
import jax.numpy as jnp
from jax import lax
from jax.experimental import pallas as pl
from jax.experimental.pallas import tpu as pltpu
from jax.experimental.pallas import tpu_sc as plsc

T = 2048
D = 2048
E = 8
K = 2
INTER = 1024
SH_INTER = 1024
P = T * K

TT = 256
NT = P // TT + E
R = NT * TT
NTP = 48

NC = 2
NS = 16
NW = NC * NS
LANES = 16

@functools.lru_cache(maxsize=None)
def _sc_mesh():
    return plsc.VectorSubcoreMesh(
        core_axis_name="c", subcore_axis_name="s",
        num_cores=NC, num_subcores=NS)


def _gate_body(x_ref, gw_ref, gb_ref, idx_ref, wt_ref):
    x = x_ref[...]
    gw = gw_ref[...]
    logits = lax.dot_general(x, gw, (((1,), (1,)), ((), ())),
                             preferred_element_type=jnp.float32)
    m = jnp.max(logits, axis=1, keepdims=True)
    ex = jnp.exp(logits - m)
    s = ex / jnp.sum(ex, axis=1, keepdims=True)
    b = s + gb_ref[...]
    iota = lax.broadcasted_iota(jnp.int32, s.shape, 1)
    v1 = jnp.max(b, axis=1, keepdims=True)
    i1 = jnp.min(jnp.where(b >= v1, iota, E), axis=1, keepdims=True)
    w1 = jnp.sum(jnp.where(iota == i1, s, 0.0), axis=1, keepdims=True)
    b2 = jnp.where(iota == i1, -jnp.inf, b)
    v2 = jnp.max(b2, axis=1, keepdims=True)
    i2 = jnp.min(jnp.where(b2 >= v2, iota, E), axis=1, keepdims=True)
    w2 = jnp.sum(jnp.where(iota == i2, s, 0.0), axis=1, keepdims=True)
    idx_ref[...] = jnp.concatenate([i1, i2], axis=1)
    wt_ref[...] = jnp.concatenate([w1, w2], axis=1)


def _gate(x, gate_w, gate_b):
    bt = 1024
    return pl.pallas_call(
        _gate_body,
        grid=(T // bt,),
        in_specs=[
            pl.BlockSpec((bt, D), lambda i: (i, 0)),
            pl.BlockSpec((E, D), lambda i: (0, 0)),
            pl.BlockSpec((1, E), lambda i: (0, 0)),
        ],
        out_specs=[
            pl.BlockSpec((bt, K), lambda i: (i, 0)),
            pl.BlockSpec((bt, K), lambda i: (i, 0)),
        ],
        out_shape=[
            jax.ShapeDtypeStruct((T, K), jnp.int32),
            jax.ShapeDtypeStruct((T, K), jnp.float32),
        ],
    )(x, gate_w, gate_b.reshape(1, E))


def _permute(v, idx):
    return lax.gather(
        v, idx[:, None],
        lax.GatherDimensionNumbers(
            offset_dims=(), collapsed_slice_dims=(0,), start_index_map=(0,)),
        (1,), mode=lax.GatherScatterMode.PROMISE_IN_BOUNDS)


def _bcast_lane(v, e):
    return _permute(v, jnp.full((LANES,), e, jnp.int32))


def _incl_scan(s, ii):
    for d in (1, 2, 4, 8):
        g = _permute(s, jnp.maximum(ii - d, 0))
        s = s + jnp.where(ii >= d, g, 0)
    return s


def _dispatch_body(eidx_hbm, gwt_hbm, st_hbm, ws_hbm, pos_hbm, te_hbm,
                   e_v, g_v, st_v, ws_v, pos_v, te_v, stv_v):
    wid = lax.axis_index("s") * NC + lax.axis_index("c")

    @pl.when(wid == 0)
    def _():
        pltpu.sync_copy(eidx_hbm, e_v)
        pltpu.sync_copy(gwt_hbm, g_v)

        zf = jnp.zeros((LANES,), jnp.float32)
        zi = jnp.zeros((LANES,), jnp.int32)
        ii = lax.iota(jnp.int32, LANES)

        def init_body(i, _):
            st_v[pl.ds(i * LANES, LANES)] = (ii + i * LANES) & (T - 1)
            ws_v[pl.ds(i * LANES, LANES)] = zf
            return 0
        lax.fori_loop(0, R // LANES, init_body, 0)

        def rank_body(i, cnt):
            v = e_v[pl.ds(i * LANES, LANES)]
            rank = zi
            for e in range(E):
                m = v == e
                sc = _incl_scan(jnp.where(m, 1, 0), ii)
                ce = _bcast_lane(cnt, e)
                rank = jnp.where(m, ce + sc - 1, rank)
                cnt = cnt + jnp.where(ii == e, _bcast_lane(sc, LANES - 1), 0)
            pos_v[pl.ds(i * LANES, LANES)] = rank
            return cnt
        cnt = lax.fori_loop(0, P // LANES, rank_body, zi)

        tt_log = TT.bit_length() - 1
        padded = ((cnt + (TT - 1)) >> tt_log) << tt_log
        starts = _incl_scan(padded, ii) - padded
        stv_v[...] = starts

        def scat_body(i, _):
            v = e_v[pl.ds(i * LANES, LANES)]
            r = pos_v[pl.ds(i * LANES, LANES)]
            pos = plsc.load_gather(stv_v, [v]) + r
            pos_v[pl.ds(i * LANES, LANES)] = pos
            tok = (ii + i * LANES) >> 1
            plsc.store_scatter(st_v, [pos], tok)
            plsc.store_scatter(ws_v, [pos], g_v[pl.ds(i * LANES, LANES)])
            return 0
        lax.fori_loop(0, P // LANES, scat_body, 0)

        tot = _bcast_lane(starts, E - 1) + _bcast_lane(padded, E - 1)
        for i in range(NTP // LANES):
            rows = (ii + i * LANES) * TT
            acc = zi
            for e in range(1, E):
                acc = acc + jnp.where(rows >= _bcast_lane(starts, e), 1, 0)
            acc = acc + jnp.where(rows >= tot, E, 0)
            te_v[pl.ds(i * LANES, LANES)] = acc

        pltpu.sync_copy(st_v, st_hbm)
        pltpu.sync_copy(ws_v, ws_hbm)
        pltpu.sync_copy(pos_v, pos_hbm)
        pltpu.sync_copy(te_v, te_hbm)


@functools.lru_cache(maxsize=None)
def _dispatch_kernel():
    return pl.kernel(
        _dispatch_body,
        out_type=(
            jax.ShapeDtypeStruct((R,), jnp.int32),
            jax.ShapeDtypeStruct((R,), jnp.float32),
            jax.ShapeDtypeStruct((P,), jnp.int32),
            jax.ShapeDtypeStruct((NTP,), jnp.int32),
        ),
        mesh=_sc_mesh(),
        scratch_types=[
            pltpu.VMEM((P,), jnp.int32),
            pltpu.VMEM((P,), jnp.float32),
            pltpu.VMEM((R,), jnp.int32),
            pltpu.VMEM((R,), jnp.float32),
            pltpu.VMEM((P,), jnp.int32),
            pltpu.VMEM((NTP,), jnp.int32),
            pltpu.VMEM((LANES,), jnp.int32),
        ],
        compiler_params=pltpu.CompilerParams(needs_layout_passes=False),
    )


@functools.lru_cache(maxsize=None)
def _make_gather(nrows):
    per_w = nrows // NW
    chunk = 16
    nch = per_w // chunk

    @functools.partial(
        pl.kernel,
        out_type=jax.ShapeDtypeStruct((nrows, D), jnp.float32),
        mesh=_sc_mesh(),
        scratch_types=[
            pltpu.VMEM((per_w,), jnp.int32),
            pltpu.VMEM((chunk, D), jnp.float32),
            pltpu.VMEM((chunk, D), jnp.float32),
            pltpu.SemaphoreType.DMA,
            pltpu.SemaphoreType.DMA,
            pltpu.SemaphoreType.DMA,
            pltpu.SemaphoreType.DMA,
        ],
        compiler_params=pltpu.CompilerParams(needs_layout_passes=False),
    )
    def gather(table_hbm, idx_hbm, out_hbm, idx_v, rows_a, rows_b,
               gsem_a, gsem_b, osem_a, osem_b):
        wid = lax.axis_index("s") * NC + lax.axis_index("c")
        base = wid * per_w
        pltpu.sync_copy(idx_hbm.at[pl.ds(base, per_w)], idx_v)
        bufs = (rows_a, rows_b)
        gsems = (gsem_a, gsem_b)
        osems = (osem_a, osem_b)
        gets = [None] * nch
        puts = [None] * nch
        gets[0] = pltpu.async_copy(
            table_hbm.at[idx_v.at[pl.ds(0, chunk)]], bufs[0], gsems[0])
        for ch in range(nch):
            if ch + 1 < nch:
                if ch >= 1:
                    puts[ch - 1].wait()
                gets[ch + 1] = pltpu.async_copy(
                    table_hbm.at[idx_v.at[pl.ds((ch + 1) * chunk, chunk)]],
                    bufs[(ch + 1) % 2], gsems[(ch + 1) % 2])
            gets[ch].wait()
            puts[ch] = pltpu.async_copy(
                bufs[ch % 2], out_hbm.at[pl.ds(base + ch * chunk, chunk)],
                osems[ch % 2])
        puts[nch - 2].wait()
        puts[nch - 1].wait()

    return gather


@functools.lru_cache(maxsize=None)
def _gather_combine_kernel():
    tok_w = T // NW
    ct = 8
    nch = tok_w // ct

    @functools.partial(
        pl.kernel,
        out_type=jax.ShapeDtypeStruct((T, D), jnp.float32),
        mesh=_sc_mesh(),
        scratch_types=[
            pltpu.VMEM((2 * ct,), jnp.int32),
            pltpu.VMEM((2 * ct, D), jnp.float32),
            pltpu.VMEM((ct, D), jnp.float32),
            pltpu.VMEM((ct, D), jnp.float32),
            pltpu.SemaphoreType.DMA,
            pltpu.SemaphoreType.DMA,
        ],
        compiler_params=pltpu.CompilerParams(needs_layout_passes=False),
    )
    def gather_combine(ys_hbm, pos_hbm, z_hbm, out_hbm,
                       pidx_v, rows_v, z_v, out_v, gsem, osem):
        wid = lax.axis_index("s") * NC + lax.axis_index("c")
        base = wid * tok_w
        for c in range(nch):
            t0 = base + c * ct
            pltpu.sync_copy(pos_hbm.at[pl.ds(2 * t0, 2 * ct)], pidx_v)
            get = pltpu.async_copy(ys_hbm.at[pidx_v], rows_v, gsem)
            pltpu.sync_copy(z_hbm.at[pl.ds(t0, ct)], z_v)
            get.wait()
            if c >= 1:
                put.wait()
            for j in range(ct):
                def dbody(k, _):
                    sl = pl.ds(k * LANES, LANES)
                    out_v[j, sl] = (z_v[j, sl] + rows_v[2 * j, sl]
                                    + rows_v[2 * j + 1, sl])
                    return 0
                lax.fori_loop(0, D // LANES, dbody, 0)
            put = pltpu.async_copy(out_v, out_hbm.at[pl.ds(t0, ct)], osem)
        put.wait()

    return gather_combine


def _ffn_body(te_ref, xs_ref, ws_ref, w1_ref, w3_ref, w2_ref, out_ref):
    i = pl.program_id(0)

    @pl.when(te_ref[i] < E)
    def _compute():
        xv = xs_ref[...]
        a = lax.dot_general(xv, w1_ref[0], (((1,), (1,)), ((), ())),
                            preferred_element_type=jnp.float32)
        b = lax.dot_general(xv, w3_ref[0], (((1,), (1,)), ((), ())),
                            preferred_element_type=jnp.float32)
        h = (a * jax.nn.sigmoid(a)) * b
        o = lax.dot_general(h, w2_ref[0], (((1,), (1,)), ((), ())),
                            preferred_element_type=jnp.float32)
        out_ref[...] = o * ws_ref[...]

    @pl.when(te_ref[i] >= E)
    def _skip():
        out_ref[...] = jnp.zeros_like(out_ref)


def _ffn(te, xs, ws, ew1, ew3, ew2):
    grid_spec = pltpu.PrefetchScalarGridSpec(
        num_scalar_prefetch=1,
        grid=(NT,),
        in_specs=[
            pl.BlockSpec((TT, D), lambda i, te: (i, 0)),
            pl.BlockSpec((TT, 1), lambda i, te: (i, 0)),
            pl.BlockSpec((1, INTER, D), lambda i, te: (te[i] & (E - 1), 0, 0)),
            pl.BlockSpec((1, INTER, D), lambda i, te: (te[i] & (E - 1), 0, 0)),
            pl.BlockSpec((1, D, INTER), lambda i, te: (te[i] & (E - 1), 0, 0)),
        ],
        out_specs=pl.BlockSpec((TT, D), lambda i, te: (i, 0)),
    )
    return pl.pallas_call(
        _ffn_body,
        grid_spec=grid_spec,
        out_shape=jax.ShapeDtypeStruct((R, D), jnp.float32),
    )(te, xs, ws, ew1, ew3, ew2)


def _shared_body(x_ref, sw1_ref, sw3_ref, sw2_ref, out_ref):
    xv = x_ref[...]
    a = lax.dot_general(xv, sw1_ref[...], (((1,), (1,)), ((), ())),
                        preferred_element_type=jnp.float32)
    b = lax.dot_general(xv, sw3_ref[...], (((1,), (1,)), ((), ())),
                        preferred_element_type=jnp.float32)
    h = (a * jax.nn.sigmoid(a)) * b
    out_ref[...] = lax.dot_general(h, sw2_ref[...], (((1,), (1,)), ((), ())),
                                   preferred_element_type=jnp.float32)


def _shared(x, sw1, sw3, sw2):
    bt = 512
    return pl.pallas_call(
        _shared_body,
        grid=(T // bt,),
        in_specs=[
            pl.BlockSpec((bt, D), lambda i: (i, 0)),
            pl.BlockSpec((SH_INTER, D), lambda i: (0, 0)),
            pl.BlockSpec((SH_INTER, D), lambda i: (0, 0)),
            pl.BlockSpec((D, SH_INTER), lambda i: (0, 0)),
        ],
        out_specs=pl.BlockSpec((bt, D), lambda i: (i, 0)),
        out_shape=jax.ShapeDtypeStruct((T, D), jnp.float32),
    )(x, sw1, sw3, sw2)


def _combine_body(z_ref, yg_ref, out_ref):
    out_ref[...] = z_ref[...] + yg_ref[:, 0, :] + yg_ref[:, 1, :]


def _combine(z, yg):
    bt = 512
    return pl.pallas_call(
        _combine_body,
        grid=(T // bt,),
        in_specs=[
            pl.BlockSpec((bt, D), lambda i: (i, 0)),
            pl.BlockSpec((bt, K, D), lambda i: (i, 0, 0)),
        ],
        out_specs=pl.BlockSpec((bt, D), lambda i: (i, 0)),
        out_shape=jax.ShapeDtypeStruct((T, D), jnp.float32),
    )(z, yg)


def kernel(x, gate_w, gate_b, ew1, ew2, ew3, sw1, sw2, sw3):
    eidx, gwt = _gate(x, gate_w, gate_b)
    st, ws, pos, te = _dispatch_kernel()(eidx.reshape(P), gwt.reshape(P))
    z = _shared(x, sw1, sw3, sw2)
    xs = _make_gather(R)(x, st)
    ys = _ffn(te, xs, ws.reshape(R, 1), ew1, ew3, ew2)
    out = _gather_combine_kernel()(ys, pos, z)
    return out.reshape(x.shape)
